# Initial kernel scaffold; baseline (speedup 1.0000x reference)
import jax
import jax.numpy as jnp
from jax.experimental import pallas as pl


def kernel(x, edge_index, edge_type, W1, Q1, K1, W2, Q2, K2, Wl, bl):
    raise NotImplementedError("write your pallas kernel here")


# speedup vs baseline: 20.4404x; 20.4404x over previous
import functools

import jax
import jax.numpy as jnp
from jax import lax
from jax.experimental import pallas as pl
from jax.experimental.pallas import tpu as pltpu
from jax.experimental.pallas import tpu_sc as plsc

N = 10000
E = 320000
IN = 128
H = 128
OUT = 128
R = 8
HP = H + 16
NC = 2
NS = 16
NW = NC * NS
PER_W = E // NW
B = 80
NB = PER_W // B
BN = 1000
ROWS_PER_SUB = 624
TAIL_ROW0 = NS * ROWS_PER_SUB
TAIL_ROWS = N - TAIL_ROW0


def _idx_body(src_ref, dst_ref, et_ref, iq_ref, ik_ref):
    et = et_ref[...]
    iq_ref[...] = dst_ref[...] * R + et
    ik_ref[...] = et * N + src_ref[...]


def _make_idx():
    shp = (625, 512)
    spec = pl.BlockSpec(shp, lambda: (0, 0))
    return pl.pallas_call(
        _idx_body,
        grid=(),
        in_specs=[spec, spec, spec],
        out_specs=[spec, spec],
        out_shape=[jax.ShapeDtypeStruct(shp, jnp.int32)] * 2,
    )


def _dense_body(x_ref, w_ref, k_ref, xw_ref):
    xw = jnp.dot(x_ref[...], w_ref[0], preferred_element_type=jnp.float32)
    kn = jnp.sum(xw * k_ref[0, 0][None, :], axis=1)
    col = lax.broadcasted_iota(jnp.int32, (BN, HP - H), 1)
    ext = jnp.where(col == 0, 1.0, jnp.where(col == 1, kn[:, None], 0.0))
    xw_ref[0] = jnp.concatenate([xw, ext], axis=1)


def _make_dense():
    return pl.pallas_call(
        _dense_body,
        grid=(R, N // BN),
        in_specs=[
            pl.BlockSpec((BN, IN), lambda r, nb: (nb, 0)),
            pl.BlockSpec((1, IN, H), lambda r, nb: (r, 0, 0)),
            pl.BlockSpec((1, 1, H), lambda r, nb: (r, 0, 0)),
        ],
        out_specs=pl.BlockSpec((1, BN, HP), lambda r, nb: (r, nb, 0)),
        out_shape=jax.ShapeDtypeStruct((R, N, HP), jnp.float32),
    )


def _qn_body(x_ref, w_ref, q_ref, qn_ref):
    wq = jnp.stack([jnp.dot(w_ref[r], q_ref[r, 0]) for r in range(R)], axis=0)
    qn_ref[...] = lax.dot_general(x_ref[...], wq, (((1,), (1,)), ((), ())),
                                  preferred_element_type=jnp.float32)


def _make_qn():
    return pl.pallas_call(
        _qn_body,
        grid=(N // BN,),
        in_specs=[
            pl.BlockSpec((BN, IN), lambda nb: (nb, 0)),
            pl.BlockSpec((R, IN, H), lambda nb: (0, 0, 0)),
            pl.BlockSpec((R, 1, H), lambda nb: (0, 0, 0)),
        ],
        out_specs=pl.BlockSpec((BN, R), lambda nb: (nb, 0)),
        out_shape=jax.ShapeDtypeStruct((N, R), jnp.float32),
    )


def _combine_body(acc_ref, h_ref):
    a = acc_ref[0] + acc_ref[1]
    h_ref[...] = jnp.maximum(a[:, :H] / (a[:, H:H + 1] + 1e-16), 0.0)


def _make_combine():
    return pl.pallas_call(
        _combine_body,
        grid=(N // BN,),
        in_specs=[pl.BlockSpec((2, BN, HP), lambda nb: (0, nb, 0))],
        out_specs=pl.BlockSpec((BN, H), lambda nb: (nb, 0)),
        out_shape=jax.ShapeDtypeStruct((N, H), jnp.float32),
    )


def _final_body(acc_ref, wl_ref, bl_ref, y_ref):
    a = acc_ref[0] + acc_ref[1]
    h = jnp.maximum(a[:, :H] / (a[:, H:H + 1] + 1e-16), 0.0)
    y_ref[...] = jnp.dot(h, wl_ref[...],
                         preferred_element_type=jnp.float32) + bl_ref[0][None, :]


def _make_final():
    return pl.pallas_call(
        _final_body,
        grid=(N // BN,),
        in_specs=[
            pl.BlockSpec((2, BN, HP), lambda nb: (0, nb, 0)),
            pl.BlockSpec((H, OUT), lambda nb: (0, 0)),
            pl.BlockSpec((1, OUT), lambda nb: (0, 0)),
        ],
        out_specs=pl.BlockSpec((BN, OUT), lambda nb: (nb, 0)),
        out_shape=jax.ShapeDtypeStruct((N, OUT), jnp.float32),
    )


def _edge_body(iq_hbm, ik_hbm, dst_hbm, qn_hbm, xw_hbm, acc_hbm,
               iqv, ikv, dstv, qv, rows, zrow, acc_sh, sem_q, sem_r):
    c = lax.axis_index("c")
    s = lax.axis_index("s")
    wid = s * NC + c
    base = wid * PER_W
    row0 = s * ROWS_PER_SUB

    for bb in range(16):
        for j in range(HP // 16):
            zrow[bb, pl.ds(j * 16, 16)] = jnp.zeros((16,), jnp.float32)
    for k in range(ROWS_PER_SUB // 16):
        pltpu.sync_copy(zrow, acc_sh.at[pl.ds(row0 + k * 16, 16)])
    @pl.when(s == 0)
    def _zero_tail():
        pltpu.sync_copy(zrow, acc_sh.at[pl.ds(TAIL_ROW0, TAIL_ROWS)])
    plsc.subcore_barrier()

    def batch(t, carry):
        off = base + t * B
        pltpu.sync_copy(iq_hbm.at[pl.ds(off, B)], iqv)
        pltpu.sync_copy(ik_hbm.at[pl.ds(off, B)], ikv)
        pltpu.sync_copy(dst_hbm.at[pl.ds(off, B)], dstv)
        pltpu.async_copy(qn_hbm.at[iqv], qv, sem_q).wait()
        pltpu.async_copy(xw_hbm.at[ikv], rows, sem_r).wait()
        for i in range(B // 16):
            bidx = lax.iota(jnp.int32, 16) + i * 16
            kch = plsc.load_gather(rows, [bidx, jnp.full((16,), H + 1, jnp.int32)])
            z = qv[pl.ds(i * 16, 16)] + kch
            z = jnp.maximum(z, 0.2 * z)
            qv[pl.ds(i * 16, 16)] = jnp.exp(z)
        def mulb(b, cc):
            wb = plsc.load_gather(qv, [jnp.full((16,), 0, jnp.int32) + b])
            for j in range(HP // 16):
                rows[b, pl.ds(j * 16, 16)] = rows[b, pl.ds(j * 16, 16)] * wb
            return cc
        lax.fori_loop(0, B, mulb, 0, unroll=4)
        pltpu.sync_copy(rows, acc_sh.at[dstv], add=True)
        return carry

    lax.fori_loop(0, NB, batch, 0)
    plsc.subcore_barrier()

    pltpu.sync_copy(acc_sh.at[pl.ds(row0, ROWS_PER_SUB)],
                    acc_hbm.at[c, pl.ds(row0, ROWS_PER_SUB)])
    @pl.when(s == 0)
    def _dump_tail():
        pltpu.sync_copy(acc_sh.at[pl.ds(TAIL_ROW0, TAIL_ROWS)],
                        acc_hbm.at[c, pl.ds(TAIL_ROW0, TAIL_ROWS)])


def _make_edge():
    mesh = plsc.VectorSubcoreMesh(core_axis_name="c", subcore_axis_name="s")
    return functools.partial(
        pl.kernel,
        out_type=jax.ShapeDtypeStruct((NC, N, HP), jnp.float32),
        mesh=mesh,
        compiler_params=pltpu.CompilerParams(use_tc_tiling_on_sc=False,
                                             needs_layout_passes=False),
        scratch_types=[
            pltpu.VMEM((B,), jnp.int32),
            pltpu.VMEM((B,), jnp.int32),
            pltpu.VMEM((B,), jnp.int32),
            pltpu.VMEM((B,), jnp.float32),
            pltpu.VMEM((B, HP), jnp.float32),
            pltpu.VMEM((16, HP), jnp.float32),
            pltpu.VMEM_SHARED((N, HP), jnp.float32),
            pltpu.SemaphoreType.DMA,
            pltpu.SemaphoreType.DMA,
        ],
    )(_edge_body)


def kernel(x, edge_index, edge_type, W1, Q1, K1, W2, Q2, K2, Wl, bl):
    src = edge_index[0].reshape(625, 512)
    dst = edge_index[1].reshape(625, 512)
    et = edge_type.reshape(625, 512)
    iq, ik = _make_idx()(src, dst, et)
    iq = iq.reshape(E)
    ik = ik.reshape(E)
    dst_flat = dst.reshape(E)

    dense = _make_dense()
    qnk = _make_qn()
    edge = _make_edge()

    xw1 = dense(x, W1, K1.reshape(R, 1, H))
    qn1 = qnk(x, W1, Q1.reshape(R, 1, H))
    acc1 = edge(iq, ik, dst_flat, qn1.reshape(N * R), xw1.reshape(R * N, HP))
    h1 = _make_combine()(acc1)

    xw2 = dense(h1, W2, K2.reshape(R, 1, H))
    qn2 = qnk(h1, W2, Q2.reshape(R, 1, H))
    acc2 = edge(iq, ik, dst_flat, qn2.reshape(N * R), xw2.reshape(R * N, HP))
    return _make_final()(acc2, Wl, bl.reshape(1, OUT))


# speedup vs baseline: 38.8961x; 1.9029x over previous
import functools

import jax
import jax.numpy as jnp
from jax import lax
from jax.experimental import pallas as pl
from jax.experimental.pallas import tpu as pltpu
from jax.experimental.pallas import tpu_sc as plsc

N = 10000
E = 320000
IN = 128
H = 128
OUT = 128
R = 8
HP = H + 16
NC = 2
NS = 16
NW = NC * NS
PER_W = E // NW
B = 80
NB = PER_W // B
BN = 1000
ROWS_PER_SUB = 624
TAIL_ROW0 = NS * ROWS_PER_SUB
TAIL_ROWS = N - TAIL_ROW0


def _idx_body(src_ref, dst_ref, et_ref, p_ref):
    iq = dst_ref[...] * R + et_ref[...]
    p_ref[...] = jnp.bitwise_or(jnp.left_shift(iq, 14), src_ref[...])


def _make_idx():
    shp = (625, 512)
    spec = pl.BlockSpec(shp, lambda: (0, 0))
    return pl.pallas_call(
        _idx_body,
        grid=(),
        in_specs=[spec, spec, spec],
        out_specs=spec,
        out_shape=jax.ShapeDtypeStruct(shp, jnp.int32),
    )


def _dense_body(x_ref, w_ref, k_ref, xw_ref):
    xw = jnp.dot(x_ref[...], w_ref[0], preferred_element_type=jnp.float32)
    kn = jnp.sum(xw * k_ref[0, 0][None, :], axis=1)
    col = lax.broadcasted_iota(jnp.int32, (BN, HP - H), 1)
    ext = jnp.where(col == 0, 1.0, jnp.where(col == 1, kn[:, None], 0.0))
    xw_ref[0] = jnp.concatenate([xw, ext], axis=1)


def _make_dense():
    return pl.pallas_call(
        _dense_body,
        grid=(R, N // BN),
        in_specs=[
            pl.BlockSpec((BN, IN), lambda r, nb: (nb, 0)),
            pl.BlockSpec((1, IN, H), lambda r, nb: (r, 0, 0)),
            pl.BlockSpec((1, 1, H), lambda r, nb: (r, 0, 0)),
        ],
        out_specs=pl.BlockSpec((1, BN, HP), lambda r, nb: (r, nb, 0)),
        out_shape=jax.ShapeDtypeStruct((R, N, HP), jnp.float32),
    )


def _qn_body(x_ref, w_ref, q_ref, qn_ref):
    wq = jnp.stack([jnp.dot(w_ref[r], q_ref[r, 0]) for r in range(R)], axis=0)
    qn_ref[...] = lax.dot_general(x_ref[...], wq, (((1,), (1,)), ((), ())),
                                  preferred_element_type=jnp.float32)


def _make_qn():
    return pl.pallas_call(
        _qn_body,
        grid=(N // BN,),
        in_specs=[
            pl.BlockSpec((BN, IN), lambda nb: (nb, 0)),
            pl.BlockSpec((R, IN, H), lambda nb: (0, 0, 0)),
            pl.BlockSpec((R, 1, H), lambda nb: (0, 0, 0)),
        ],
        out_specs=pl.BlockSpec((BN, R), lambda nb: (nb, 0)),
        out_shape=jax.ShapeDtypeStruct((N, R), jnp.float32),
    )


def _combine_body(acc_ref, h_ref):
    a = acc_ref[0] + acc_ref[1]
    h_ref[...] = jnp.maximum(a[:, :H] / (a[:, H:H + 1] + 1e-16), 0.0)


def _make_combine():
    return pl.pallas_call(
        _combine_body,
        grid=(N // BN,),
        in_specs=[pl.BlockSpec((2, BN, HP), lambda nb: (0, nb, 0))],
        out_specs=pl.BlockSpec((BN, H), lambda nb: (nb, 0)),
        out_shape=jax.ShapeDtypeStruct((N, H), jnp.float32),
    )


def _final_body(acc_ref, wl_ref, bl_ref, y_ref):
    a = acc_ref[0] + acc_ref[1]
    h = jnp.maximum(a[:, :H] / (a[:, H:H + 1] + 1e-16), 0.0)
    y_ref[...] = jnp.dot(h, wl_ref[...],
                         preferred_element_type=jnp.float32) + bl_ref[0][None, :]


def _make_final():
    return pl.pallas_call(
        _final_body,
        grid=(N // BN,),
        in_specs=[
            pl.BlockSpec((2, BN, HP), lambda nb: (0, nb, 0)),
            pl.BlockSpec((H, OUT), lambda nb: (0, 0)),
            pl.BlockSpec((1, OUT), lambda nb: (0, 0)),
        ],
        out_specs=pl.BlockSpec((BN, OUT), lambda nb: (nb, 0)),
        out_shape=jax.ShapeDtypeStruct((N, OUT), jnp.float32),
    )


def _edge_body(p_hbm, qn_hbm, xw_hbm, acc_hbm,
               packed, iqb, ikb, dstb, qa, rows2, wv, acc_sh,
               sem_q0, sem_q1, sem_r0, sem_r1):
    c = lax.axis_index("c")
    s = lax.axis_index("s")
    wid = s * NC + c
    brow0 = wid * NB
    row0 = s * ROWS_PER_SUB

    for sl in range(2):
        def zb(b, cc):
            for j in range(HP // 16):
                rows2[sl, b, pl.ds(j * 16, 16)] = jnp.zeros((16,), jnp.float32)
            return cc
        lax.fori_loop(0, B, zb, 0)
    for k in range(ROWS_PER_SUB // B):
        pltpu.sync_copy(rows2.at[0], acc_sh.at[pl.ds(row0 + k * B, B)])
    pltpu.sync_copy(rows2.at[0].at[pl.ds(0, ROWS_PER_SUB % B)],
                    acc_sh.at[pl.ds(row0 + (ROWS_PER_SUB // B) * B,
                                    ROWS_PER_SUB % B)])
    @pl.when(s == 0)
    def _zero_tail():
        pltpu.sync_copy(rows2.at[0].at[pl.ds(0, TAIL_ROWS)],
                        acc_sh.at[pl.ds(TAIL_ROW0, TAIL_ROWS)])

    pltpu.sync_copy(p_hbm.at[pl.ds(brow0, NB)], packed)
    plsc.subcore_barrier()

    sem_q = (sem_q0, sem_q1)
    sem_r = (sem_r0, sem_r1)

    def fire(t, slot):
        for i in range(B // 16):
            pch = packed[t, pl.ds(i * 16, 16)]
            iqc = lax.shift_right_logical(pch, 14)
            iqb[slot, pl.ds(i * 16, 16)] = iqc
            dstb[slot, pl.ds(i * 16, 16)] = lax.shift_right_logical(pch, 17)
            ikb[slot, pl.ds(i * 16, 16)] = (
                jnp.bitwise_and(iqc, R - 1) * N + jnp.bitwise_and(pch, 16383))
        pltpu.async_copy(qn_hbm.at[iqb.at[slot]], qa.at[slot], sem_q[slot])
        pltpu.async_copy(xw_hbm.at[ikb.at[slot]], rows2.at[slot], sem_r[slot])

    def process(t, slot):
        pltpu.make_async_copy(qn_hbm.at[iqb.at[slot]], qa.at[slot],
                              sem_q[slot]).wait()
        pltpu.make_async_copy(xw_hbm.at[ikb.at[slot]], rows2.at[slot],
                              sem_r[slot]).wait()
        rr = rows2.at[slot]
        for i in range(B // 16):
            bidx = lax.iota(jnp.int32, 16) + i * 16
            kch = plsc.load_gather(rr, [bidx, jnp.full((16,), H + 1, jnp.int32)])
            z = qa[slot, pl.ds(i * 16, 16)] + kch
            z = jnp.maximum(z, 0.2 * z)
            wv[pl.ds(i * 16, 16)] = jnp.exp(z)
        def mulb(b, cc):
            wb = plsc.load_gather(wv, [jnp.full((16,), 0, jnp.int32) + b])
            for j in range(HP // 16):
                rr[b, pl.ds(j * 16, 16)] = rr[b, pl.ds(j * 16, 16)] * wb
            return cc
        lax.fori_loop(0, B, mulb, 0, unroll=4)
        pltpu.sync_copy(rr, acc_sh.at[dstb.at[slot]], add=True)

    fire(0, 0)
    def body(u, carry):
        t0 = 2 * u
        @pl.when(t0 + 1 < NB)
        def _f1():
            fire(t0 + 1, 1)
        process(t0, 0)
        @pl.when(t0 + 2 < NB)
        def _f0():
            fire(t0 + 2, 0)
        @pl.when(t0 + 1 < NB)
        def _p1():
            process(t0 + 1, 1)
        return carry
    lax.fori_loop(0, (NB + 1) // 2, body, 0)
    plsc.subcore_barrier()

    pltpu.sync_copy(acc_sh.at[pl.ds(row0, ROWS_PER_SUB)],
                    acc_hbm.at[c, pl.ds(row0, ROWS_PER_SUB)])
    @pl.when(s == 0)
    def _dump_tail():
        pltpu.sync_copy(acc_sh.at[pl.ds(TAIL_ROW0, TAIL_ROWS)],
                        acc_hbm.at[c, pl.ds(TAIL_ROW0, TAIL_ROWS)])


def _make_edge():
    mesh = plsc.VectorSubcoreMesh(core_axis_name="c", subcore_axis_name="s")
    return functools.partial(
        pl.kernel,
        out_type=jax.ShapeDtypeStruct((NC, N, HP), jnp.float32),
        mesh=mesh,
        compiler_params=pltpu.CompilerParams(use_tc_tiling_on_sc=False,
                                             needs_layout_passes=False),
        scratch_types=[
            pltpu.VMEM((NB, B), jnp.int32),
            pltpu.VMEM((2, B), jnp.int32),
            pltpu.VMEM((2, B), jnp.int32),
            pltpu.VMEM((2, B), jnp.int32),
            pltpu.VMEM((2, B), jnp.float32),
            pltpu.VMEM((2, B, HP), jnp.float32),
            pltpu.VMEM((B,), jnp.float32),
            pltpu.VMEM_SHARED((N, HP), jnp.float32),
            pltpu.SemaphoreType.DMA,
            pltpu.SemaphoreType.DMA,
            pltpu.SemaphoreType.DMA,
            pltpu.SemaphoreType.DMA,
        ],
    )(_edge_body)


def kernel(x, edge_index, edge_type, W1, Q1, K1, W2, Q2, K2, Wl, bl):
    src = edge_index[0].reshape(625, 512)
    dst = edge_index[1].reshape(625, 512)
    et = edge_type.reshape(625, 512)
    packed = _make_idx()(src, dst, et).reshape(E // B, B)

    dense = _make_dense()
    qnk = _make_qn()
    edge = _make_edge()

    xw1 = dense(x, W1, K1.reshape(R, 1, H))
    qn1 = qnk(x, W1, Q1.reshape(R, 1, H))
    acc1 = edge(packed, qn1.reshape(N * R), xw1.reshape(R * N, HP))
    h1 = _make_combine()(acc1)

    xw2 = dense(h1, W2, K2.reshape(R, 1, H))
    qn2 = qnk(h1, W2, Q2.reshape(R, 1, H))
    acc2 = edge(packed, qn2.reshape(N * R), xw2.reshape(R * N, HP))
    return _make_final()(acc2, Wl, bl.reshape(1, OUT))


# speedup vs baseline: 50.8242x; 1.3067x over previous
import functools

import jax
import jax.numpy as jnp
from jax import lax
from jax.experimental import pallas as pl
from jax.experimental.pallas import tpu as pltpu
from jax.experimental.pallas import tpu_sc as plsc

N = 10000
E = 320000
IN = 128
H = 128
OUT = 128
R = 8
EW = 16
NC = 2
NS = 16
NW = NC * NS
PER_W = E // NW
B = 80
NB = PER_W // B
BN = 1000
ROWS_PER_SUB = 624
TAIL_ROW0 = NS * ROWS_PER_SUB
TAIL_ROWS = N - TAIL_ROW0


def _idx_body(src_ref, dst_ref, et_ref, p_ref):
    iq = dst_ref[...] * R + et_ref[...]
    p_ref[...] = jnp.bitwise_or(jnp.left_shift(iq, 14), src_ref[...])


def _make_idx():
    shp = (625, 512)
    spec = pl.BlockSpec(shp, lambda: (0, 0))
    return pl.pallas_call(
        _idx_body,
        grid=(),
        in_specs=[spec, spec, spec],
        out_specs=spec,
        out_shape=jax.ShapeDtypeStruct(shp, jnp.int32),
    )


def _dense_body(x_ref, w_ref, xw_ref):
    xw_ref[...] = jnp.dot(x_ref[...], w_ref[0],
                          preferred_element_type=jnp.float32)


def _make_dense():
    nblk = N // BN
    return pl.pallas_call(
        _dense_body,
        grid=(R, nblk),
        in_specs=[
            pl.BlockSpec((BN, IN), lambda r, nb: (nb, 0)),
            pl.BlockSpec((1, IN, H), lambda r, nb: (r, 0, 0)),
        ],
        out_specs=pl.BlockSpec((BN, H), lambda r, nb: (r * nblk + nb, 0)),
        out_shape=jax.ShapeDtypeStruct((R * N, H), jnp.float32),
    )


def _qk_body(x_ref, w_ref, q_ref, k_ref, qn_ref, kn_ref):
    wq = jnp.stack([jnp.dot(w_ref[r], q_ref[r, 0]) for r in range(R)], axis=0)
    wk = jnp.stack([jnp.dot(w_ref[r], k_ref[r, 0]) for r in range(R)], axis=0)
    dn = (((1,), (1,)), ((), ()))
    qn_ref[...] = lax.dot_general(x_ref[...], wq, dn,
                                  preferred_element_type=jnp.float32)
    kn_ref[...] = lax.dot_general(x_ref[...], wk, dn,
                                  preferred_element_type=jnp.float32)


def _make_qk():
    return pl.pallas_call(
        _qk_body,
        grid=(N // BN,),
        in_specs=[
            pl.BlockSpec((BN, IN), lambda nb: (nb, 0)),
            pl.BlockSpec((R, IN, H), lambda nb: (0, 0, 0)),
            pl.BlockSpec((R, 1, H), lambda nb: (0, 0, 0)),
            pl.BlockSpec((R, 1, H), lambda nb: (0, 0, 0)),
        ],
        out_specs=[
            pl.BlockSpec((BN, R), lambda nb: (nb, 0)),
            pl.BlockSpec((BN, R), lambda nb: (nb, 0)),
        ],
        out_shape=[jax.ShapeDtypeStruct((N, R), jnp.float32)] * 2,
    )


def _combine_body(acc_ref, ext_ref, h_ref):
    a = acc_ref[0] + acc_ref[1]
    den = ext_ref[0, :, 0] + ext_ref[1, :, 0]
    h_ref[...] = jnp.maximum(a / (den[:, None] + 1e-16), 0.0)


def _make_combine():
    return pl.pallas_call(
        _combine_body,
        grid=(N // BN,),
        in_specs=[
            pl.BlockSpec((2, BN, H), lambda nb: (0, nb, 0)),
            pl.BlockSpec((2, BN, EW), lambda nb: (0, nb, 0)),
        ],
        out_specs=pl.BlockSpec((BN, H), lambda nb: (nb, 0)),
        out_shape=jax.ShapeDtypeStruct((N, H), jnp.float32),
    )


def _final_body(acc_ref, ext_ref, wl_ref, bl_ref, y_ref):
    a = acc_ref[0] + acc_ref[1]
    den = ext_ref[0, :, 0] + ext_ref[1, :, 0]
    h = jnp.maximum(a / (den[:, None] + 1e-16), 0.0)
    y_ref[...] = jnp.dot(h, wl_ref[...],
                         preferred_element_type=jnp.float32) + bl_ref[0][None, :]


def _make_final():
    return pl.pallas_call(
        _final_body,
        grid=(N // BN,),
        in_specs=[
            pl.BlockSpec((2, BN, H), lambda nb: (0, nb, 0)),
            pl.BlockSpec((2, BN, EW), lambda nb: (0, nb, 0)),
            pl.BlockSpec((H, OUT), lambda nb: (0, 0)),
            pl.BlockSpec((1, OUT), lambda nb: (0, 0)),
        ],
        out_specs=pl.BlockSpec((BN, OUT), lambda nb: (nb, 0)),
        out_shape=jax.ShapeDtypeStruct((N, OUT), jnp.float32),
    )


def _edge_body(p_hbm, qn_hbm, kn_hbm, xw_hbm, acc_hbm, ext_hbm,
               packed, iqb, isb, ikb, dstb, qa, ka, wv, exb, rows2,
               acc_sh, ext_sh,
               sem_q0, sem_q1, sem_k0, sem_k1, sem_r0, sem_r1):
    c = lax.axis_index("c")
    s = lax.axis_index("s")
    wid = s * NC + c
    brow0 = wid * NB
    row0 = s * ROWS_PER_SUB

    for sl in range(2):
        def zb(b, cc):
            for j in range(H // 16):
                rows2[sl, b, pl.ds(j * 16, 16)] = jnp.zeros((16,), jnp.float32)
            return cc
        lax.fori_loop(0, B, zb, 0)
    def ze(b, cc):
        exb[b, pl.ds(0, 16)] = jnp.zeros((16,), jnp.float32)
        return cc
    lax.fori_loop(0, B, ze, 0)
    for k in range(ROWS_PER_SUB // B):
        pltpu.sync_copy(rows2.at[0], acc_sh.at[pl.ds(row0 + k * B, B)])
        pltpu.sync_copy(exb, ext_sh.at[pl.ds(row0 + k * B, B)])
    rem = ROWS_PER_SUB % B
    pltpu.sync_copy(rows2.at[0].at[pl.ds(0, rem)],
                    acc_sh.at[pl.ds(row0 + (ROWS_PER_SUB // B) * B, rem)])
    pltpu.sync_copy(exb.at[pl.ds(0, rem)],
                    ext_sh.at[pl.ds(row0 + (ROWS_PER_SUB // B) * B, rem)])
    @pl.when(s == 0)
    def _zero_tail():
        pltpu.sync_copy(rows2.at[0].at[pl.ds(0, TAIL_ROWS)],
                        acc_sh.at[pl.ds(TAIL_ROW0, TAIL_ROWS)])
        pltpu.sync_copy(exb.at[pl.ds(0, TAIL_ROWS)],
                        ext_sh.at[pl.ds(TAIL_ROW0, TAIL_ROWS)])

    pltpu.sync_copy(p_hbm.at[pl.ds(brow0, NB)], packed)
    plsc.subcore_barrier()

    sem_q = (sem_q0, sem_q1)
    sem_k = (sem_k0, sem_k1)
    sem_r = (sem_r0, sem_r1)

    def fire(t, slot):
        for i in range(B // 16):
            sl16 = pl.ds(i * 16, 16)
            pch = packed[t, sl16]
            iqc = lax.shift_right_logical(pch, 14)
            srcv = jnp.bitwise_and(pch, 16383)
            etc = jnp.bitwise_and(iqc, R - 1)
            iqb[slot, sl16] = iqc
            dstb[slot, sl16] = lax.shift_right_logical(pch, 17)
            isb[slot, sl16] = srcv * R + etc
            ikb[slot, sl16] = etc * N + srcv
        pltpu.async_copy(qn_hbm.at[iqb.at[slot]], qa.at[slot], sem_q[slot])
        pltpu.async_copy(kn_hbm.at[isb.at[slot]], ka.at[slot], sem_k[slot])
        pltpu.async_copy(xw_hbm.at[ikb.at[slot]], rows2.at[slot], sem_r[slot])

    def process(t, slot):
        pltpu.make_async_copy(qn_hbm.at[iqb.at[slot]], qa.at[slot],
                              sem_q[slot]).wait()
        pltpu.make_async_copy(kn_hbm.at[isb.at[slot]], ka.at[slot],
                              sem_k[slot]).wait()
        pltpu.make_async_copy(xw_hbm.at[ikb.at[slot]], rows2.at[slot],
                              sem_r[slot]).wait()
        rr = rows2.at[slot]
        for i in range(B // 16):
            sl16 = pl.ds(i * 16, 16)
            bidx = lax.iota(jnp.int32, 16) + i * 16
            z = qa[slot, sl16] + ka[slot, sl16]
            z = jnp.maximum(z, 0.2 * z)
            w = jnp.exp(z)
            wv[sl16] = w
            plsc.store_scatter(exb, [bidx, jnp.zeros((16,), jnp.int32)], w)
        def mulb(b, cc):
            wb = plsc.load_gather(wv, [jnp.full((16,), 0, jnp.int32) + b])
            for j in range(H // 16):
                rr[b, pl.ds(j * 16, 16)] = rr[b, pl.ds(j * 16, 16)] * wb
            return cc
        lax.fori_loop(0, B, mulb, 0, unroll=8)
        pltpu.sync_copy(rr, acc_sh.at[dstb.at[slot]], add=True)
        pltpu.sync_copy(exb, ext_sh.at[dstb.at[slot]], add=True)

    fire(0, 0)
    def body(u, carry):
        t0 = 2 * u
        @pl.when(t0 + 1 < NB)
        def _f1():
            fire(t0 + 1, 1)
        process(t0, 0)
        @pl.when(t0 + 2 < NB)
        def _f0():
            fire(t0 + 2, 0)
        @pl.when(t0 + 1 < NB)
        def _p1():
            process(t0 + 1, 1)
        return carry
    lax.fori_loop(0, (NB + 1) // 2, body, 0)
    plsc.subcore_barrier()

    pltpu.sync_copy(acc_sh.at[pl.ds(row0, ROWS_PER_SUB)],
                    acc_hbm.at[c, pl.ds(row0, ROWS_PER_SUB)])
    pltpu.sync_copy(ext_sh.at[pl.ds(row0, ROWS_PER_SUB)],
                    ext_hbm.at[c, pl.ds(row0, ROWS_PER_SUB)])
    @pl.when(s == 0)
    def _dump_tail():
        pltpu.sync_copy(acc_sh.at[pl.ds(TAIL_ROW0, TAIL_ROWS)],
                        acc_hbm.at[c, pl.ds(TAIL_ROW0, TAIL_ROWS)])
        pltpu.sync_copy(ext_sh.at[pl.ds(TAIL_ROW0, TAIL_ROWS)],
                        ext_hbm.at[c, pl.ds(TAIL_ROW0, TAIL_ROWS)])


def _make_edge():
    mesh = plsc.VectorSubcoreMesh(core_axis_name="c", subcore_axis_name="s")
    return functools.partial(
        pl.kernel,
        out_type=[
            jax.ShapeDtypeStruct((NC, N, H), jnp.float32),
            jax.ShapeDtypeStruct((NC, N, EW), jnp.float32),
        ],
        mesh=mesh,
        compiler_params=pltpu.CompilerParams(use_tc_tiling_on_sc=False,
                                             needs_layout_passes=False),
        scratch_types=[
            pltpu.VMEM((NB, B), jnp.int32),
            pltpu.VMEM((2, B), jnp.int32),
            pltpu.VMEM((2, B), jnp.int32),
            pltpu.VMEM((2, B), jnp.int32),
            pltpu.VMEM((2, B), jnp.int32),
            pltpu.VMEM((2, B), jnp.float32),
            pltpu.VMEM((2, B), jnp.float32),
            pltpu.VMEM((B,), jnp.float32),
            pltpu.VMEM((B, EW), jnp.float32),
            pltpu.VMEM((2, B, H), jnp.float32),
            pltpu.VMEM_SHARED((N, H), jnp.float32),
            pltpu.VMEM_SHARED((N, EW), jnp.float32),
            pltpu.SemaphoreType.DMA,
            pltpu.SemaphoreType.DMA,
            pltpu.SemaphoreType.DMA,
            pltpu.SemaphoreType.DMA,
            pltpu.SemaphoreType.DMA,
            pltpu.SemaphoreType.DMA,
        ],
    )(_edge_body)


def kernel(x, edge_index, edge_type, W1, Q1, K1, W2, Q2, K2, Wl, bl):
    src = edge_index[0].reshape(625, 512)
    dst = edge_index[1].reshape(625, 512)
    et = edge_type.reshape(625, 512)
    packed = _make_idx()(src, dst, et).reshape(E // B, B)

    dense = _make_dense()
    qk = _make_qk()
    edge = _make_edge()
    combine = _make_combine()

    xw1 = dense(x, W1)
    qn1, kn1 = qk(x, W1, Q1.reshape(R, 1, H), K1.reshape(R, 1, H))
    acc1, ext1 = edge(packed, qn1.reshape(N * R), kn1.reshape(N * R), xw1)
    h1 = combine(acc1, ext1)

    xw2 = dense(h1, W2)
    qn2, kn2 = qk(h1, W2, Q2.reshape(R, 1, H), K2.reshape(R, 1, H))
    acc2, ext2 = edge(packed, qn2.reshape(N * R), kn2.reshape(N * R), xw2)
    return _make_final()(acc2, ext2, Wl, bl.reshape(1, OUT))


# speedup vs baseline: 53.3410x; 1.0495x over previous
import functools

import jax
import jax.numpy as jnp
from jax import lax
from jax.experimental import pallas as pl
from jax.experimental.pallas import tpu as pltpu
from jax.experimental.pallas import tpu_sc as plsc

N = 10000
E = 320000
IN = 128
H = 128
OUT = 128
R = 8
EW = 16
NC = 2
NS = 16
NW = NC * NS
PER_W = E // NW
B = 80
NB = PER_W // B
BN = 1000
ROWS_PER_SUB = 624
TAIL_ROW0 = NS * ROWS_PER_SUB
TAIL_ROWS = N - TAIL_ROW0


def _idx_body(src_ref, dst_ref, et_ref, p_ref):
    iq = dst_ref[...] * R + et_ref[...]
    p_ref[...] = jnp.bitwise_or(jnp.left_shift(iq, 14), src_ref[...])


def _make_idx():
    shp = (625, 512)
    spec = pl.BlockSpec(shp, lambda: (0, 0))
    return pl.pallas_call(
        _idx_body,
        grid=(),
        in_specs=[spec, spec, spec],
        out_specs=spec,
        out_shape=jax.ShapeDtypeStruct(shp, jnp.int32),
    )


def _dense_body(x_ref, w_ref, xw_ref):
    xw_ref[...] = jnp.dot(x_ref[...], w_ref[0],
                          preferred_element_type=jnp.float32)


def _make_dense():
    nblk = N // BN
    return pl.pallas_call(
        _dense_body,
        grid=(nblk, R),
        in_specs=[
            pl.BlockSpec((BN, IN), lambda nb, r: (nb, 0)),
            pl.BlockSpec((1, IN, H), lambda nb, r: (r, 0, 0)),
        ],
        out_specs=pl.BlockSpec((BN, H), lambda nb, r: (r * nblk + nb, 0)),
        out_shape=jax.ShapeDtypeStruct((R * N, H), jnp.float32),
    )


def _qk_body(x_ref, w_ref, q_ref, k_ref, qn_ref, kn_ref):
    wq = jnp.stack([jnp.dot(w_ref[r], q_ref[r, 0]) for r in range(R)], axis=0)
    wk = jnp.stack([jnp.dot(w_ref[r], k_ref[r, 0]) for r in range(R)], axis=0)
    dn = (((1,), (1,)), ((), ()))
    qn_ref[...] = lax.dot_general(x_ref[...], wq, dn,
                                  preferred_element_type=jnp.float32)
    kn_ref[...] = lax.dot_general(x_ref[...], wk, dn,
                                  preferred_element_type=jnp.float32)


def _make_qk():
    return pl.pallas_call(
        _qk_body,
        grid=(N // BN,),
        in_specs=[
            pl.BlockSpec((BN, IN), lambda nb: (nb, 0)),
            pl.BlockSpec((R, IN, H), lambda nb: (0, 0, 0)),
            pl.BlockSpec((R, 1, H), lambda nb: (0, 0, 0)),
            pl.BlockSpec((R, 1, H), lambda nb: (0, 0, 0)),
        ],
        out_specs=[
            pl.BlockSpec((BN, R), lambda nb: (nb, 0)),
            pl.BlockSpec((BN, R), lambda nb: (nb, 0)),
        ],
        out_shape=[jax.ShapeDtypeStruct((N, R), jnp.float32)] * 2,
    )


def _combine_body(acc_ref, ext_ref, h_ref):
    a = acc_ref[0] + acc_ref[1]
    den = ext_ref[0, :, 0] + ext_ref[1, :, 0]
    h_ref[...] = jnp.maximum(a / (den[:, None] + 1e-16), 0.0)


def _make_combine():
    return pl.pallas_call(
        _combine_body,
        grid=(N // BN,),
        in_specs=[
            pl.BlockSpec((2, BN, H), lambda nb: (0, nb, 0)),
            pl.BlockSpec((2, BN, EW), lambda nb: (0, nb, 0)),
        ],
        out_specs=pl.BlockSpec((BN, H), lambda nb: (nb, 0)),
        out_shape=jax.ShapeDtypeStruct((N, H), jnp.float32),
    )


def _final_body(acc_ref, ext_ref, wl_ref, bl_ref, y_ref):
    a = acc_ref[0] + acc_ref[1]
    den = ext_ref[0, :, 0] + ext_ref[1, :, 0]
    h = jnp.maximum(a / (den[:, None] + 1e-16), 0.0)
    y_ref[...] = jnp.dot(h, wl_ref[...],
                         preferred_element_type=jnp.float32) + bl_ref[0][None, :]


def _make_final():
    return pl.pallas_call(
        _final_body,
        grid=(N // BN,),
        in_specs=[
            pl.BlockSpec((2, BN, H), lambda nb: (0, nb, 0)),
            pl.BlockSpec((2, BN, EW), lambda nb: (0, nb, 0)),
            pl.BlockSpec((H, OUT), lambda nb: (0, 0)),
            pl.BlockSpec((1, OUT), lambda nb: (0, 0)),
        ],
        out_specs=pl.BlockSpec((BN, OUT), lambda nb: (nb, 0)),
        out_shape=jax.ShapeDtypeStruct((N, OUT), jnp.float32),
    )


def _edge_body(p_hbm, qn_hbm, kn_hbm, xw_hbm, acc_hbm, ext_hbm,
               packed, iqb, isb, ikb, dstb, qa, ka, wv, exb, rows2,
               acc_sh, ext_sh,
               sem_q0, sem_q1, sem_k0, sem_k1, sem_r0, sem_r1,
               sem_s0, sem_s1, sem_e0, sem_e1):
    c = lax.axis_index("c")
    s = lax.axis_index("s")
    wid = s * NC + c
    brow0 = wid * NB
    row0 = s * ROWS_PER_SUB

    for sl in range(2):
        def zb(b, cc):
            for j in range(H // 16):
                rows2[sl, b, pl.ds(j * 16, 16)] = jnp.zeros((16,), jnp.float32)
            return cc
        lax.fori_loop(0, B, zb, 0)
    for sl in range(2):
        def ze(b, cc):
            exb[sl, b, pl.ds(0, 16)] = jnp.zeros((16,), jnp.float32)
            return cc
        lax.fori_loop(0, B, ze, 0)
    for k in range(ROWS_PER_SUB // B):
        pltpu.sync_copy(rows2.at[0], acc_sh.at[pl.ds(row0 + k * B, B)])
        pltpu.sync_copy(exb.at[0], ext_sh.at[pl.ds(row0 + k * B, B)])
    rem = ROWS_PER_SUB % B
    pltpu.sync_copy(rows2.at[0].at[pl.ds(0, rem)],
                    acc_sh.at[pl.ds(row0 + (ROWS_PER_SUB // B) * B, rem)])
    pltpu.sync_copy(exb.at[0].at[pl.ds(0, rem)],
                    ext_sh.at[pl.ds(row0 + (ROWS_PER_SUB // B) * B, rem)])
    @pl.when(s == 0)
    def _zero_tail():
        pltpu.sync_copy(rows2.at[0].at[pl.ds(0, TAIL_ROWS)],
                        acc_sh.at[pl.ds(TAIL_ROW0, TAIL_ROWS)])
        pltpu.sync_copy(exb.at[0].at[pl.ds(0, TAIL_ROWS)],
                        ext_sh.at[pl.ds(TAIL_ROW0, TAIL_ROWS)])

    pltpu.sync_copy(p_hbm.at[pl.ds(brow0, NB)], packed)
    plsc.subcore_barrier()

    sem_q = (sem_q0, sem_q1)
    sem_k = (sem_k0, sem_k1)
    sem_r = (sem_r0, sem_r1)
    sem_s = (sem_s0, sem_s1)
    sem_e = (sem_e0, sem_e1)

    def fire(t, slot):
        @pl.when(t >= 2)
        def _drain():
            pltpu.make_async_copy(rows2.at[slot], acc_sh.at[dstb.at[slot]],
                                  sem_s[slot]).wait()
            pltpu.make_async_copy(exb.at[slot], ext_sh.at[dstb.at[slot]],
                                  sem_e[slot]).wait()
        for i in range(B // 16):
            sl16 = pl.ds(i * 16, 16)
            pch = packed[t, sl16]
            iqc = lax.shift_right_logical(pch, 14)
            srcv = jnp.bitwise_and(pch, 16383)
            etc = jnp.bitwise_and(iqc, R - 1)
            iqb[slot, sl16] = iqc
            dstb[slot, sl16] = lax.shift_right_logical(pch, 17)
            isb[slot, sl16] = srcv * R + etc
            ikb[slot, sl16] = etc * N + srcv
        pltpu.async_copy(qn_hbm.at[iqb.at[slot]], qa.at[slot], sem_q[slot])
        pltpu.async_copy(kn_hbm.at[isb.at[slot]], ka.at[slot], sem_k[slot])
        pltpu.async_copy(xw_hbm.at[ikb.at[slot]], rows2.at[slot], sem_r[slot])

    def process(t, slot):
        pltpu.make_async_copy(qn_hbm.at[iqb.at[slot]], qa.at[slot],
                              sem_q[slot]).wait()
        pltpu.make_async_copy(kn_hbm.at[isb.at[slot]], ka.at[slot],
                              sem_k[slot]).wait()
        pltpu.make_async_copy(xw_hbm.at[ikb.at[slot]], rows2.at[slot],
                              sem_r[slot]).wait()
        rr = rows2.at[slot]
        for i in range(B // 16):
            sl16 = pl.ds(i * 16, 16)
            bidx = lax.iota(jnp.int32, 16) + i * 16
            z = qa[slot, sl16] + ka[slot, sl16]
            z = jnp.maximum(z, 0.2 * z)
            w = jnp.exp(z)
            wv[sl16] = w
            plsc.store_scatter(exb.at[slot], [bidx, jnp.zeros((16,), jnp.int32)], w)
        def mulb(b, cc):
            wb = plsc.load_gather(wv, [jnp.full((16,), 0, jnp.int32) + b])
            for j in range(H // 16):
                rr[b, pl.ds(j * 16, 16)] = rr[b, pl.ds(j * 16, 16)] * wb
            return cc
        lax.fori_loop(0, B, mulb, 0, unroll=8)
        pltpu.async_copy(rr, acc_sh.at[dstb.at[slot]], sem_s[slot], add=True)
        pltpu.async_copy(exb.at[slot], ext_sh.at[dstb.at[slot]], sem_e[slot],
                         add=True)

    fire(0, 0)
    def body(u, carry):
        t0 = 2 * u
        @pl.when(t0 + 1 < NB)
        def _f1():
            fire(t0 + 1, 1)
        process(t0, 0)
        @pl.when(t0 + 2 < NB)
        def _f0():
            fire(t0 + 2, 0)
        @pl.when(t0 + 1 < NB)
        def _p1():
            process(t0 + 1, 1)
        return carry
    lax.fori_loop(0, (NB + 1) // 2, body, 0)
    for slot in range(2):
        pltpu.make_async_copy(rows2.at[slot], acc_sh.at[dstb.at[slot]],
                              sem_s[slot]).wait()
        pltpu.make_async_copy(exb.at[slot], ext_sh.at[dstb.at[slot]],
                              sem_e[slot]).wait()
    plsc.subcore_barrier()

    pltpu.sync_copy(acc_sh.at[pl.ds(row0, ROWS_PER_SUB)],
                    acc_hbm.at[c, pl.ds(row0, ROWS_PER_SUB)])
    pltpu.sync_copy(ext_sh.at[pl.ds(row0, ROWS_PER_SUB)],
                    ext_hbm.at[c, pl.ds(row0, ROWS_PER_SUB)])
    @pl.when(s == 0)
    def _dump_tail():
        pltpu.sync_copy(acc_sh.at[pl.ds(TAIL_ROW0, TAIL_ROWS)],
                        acc_hbm.at[c, pl.ds(TAIL_ROW0, TAIL_ROWS)])
        pltpu.sync_copy(ext_sh.at[pl.ds(TAIL_ROW0, TAIL_ROWS)],
                        ext_hbm.at[c, pl.ds(TAIL_ROW0, TAIL_ROWS)])


def _make_edge():
    mesh = plsc.VectorSubcoreMesh(core_axis_name="c", subcore_axis_name="s")
    return functools.partial(
        pl.kernel,
        out_type=[
            jax.ShapeDtypeStruct((NC, N, H), jnp.float32),
            jax.ShapeDtypeStruct((NC, N, EW), jnp.float32),
        ],
        mesh=mesh,
        compiler_params=pltpu.CompilerParams(use_tc_tiling_on_sc=False,
                                             needs_layout_passes=False),
        scratch_types=[
            pltpu.VMEM((NB, B), jnp.int32),
            pltpu.VMEM((2, B), jnp.int32),
            pltpu.VMEM((2, B), jnp.int32),
            pltpu.VMEM((2, B), jnp.int32),
            pltpu.VMEM((2, B), jnp.int32),
            pltpu.VMEM((2, B), jnp.float32),
            pltpu.VMEM((2, B), jnp.float32),
            pltpu.VMEM((B,), jnp.float32),
            pltpu.VMEM((2, B, EW), jnp.float32),
            pltpu.VMEM((2, B, H), jnp.float32),
            pltpu.VMEM_SHARED((N, H), jnp.float32),
            pltpu.VMEM_SHARED((N, EW), jnp.float32),
        ] + [pltpu.SemaphoreType.DMA] * 10,
    )(_edge_body)


def kernel(x, edge_index, edge_type, W1, Q1, K1, W2, Q2, K2, Wl, bl):
    src = edge_index[0].reshape(625, 512)
    dst = edge_index[1].reshape(625, 512)
    et = edge_type.reshape(625, 512)
    packed = _make_idx()(src, dst, et).reshape(E // B, B)

    dense = _make_dense()
    qk = _make_qk()
    edge = _make_edge()
    combine = _make_combine()

    xw1 = dense(x, W1)
    qn1, kn1 = qk(x, W1, Q1.reshape(R, 1, H), K1.reshape(R, 1, H))
    acc1, ext1 = edge(packed, qn1.reshape(N * R), kn1.reshape(N * R), xw1)
    h1 = combine(acc1, ext1)

    xw2 = dense(h1, W2)
    qn2, kn2 = qk(h1, W2, Q2.reshape(R, 1, H), K2.reshape(R, 1, H))
    acc2, ext2 = edge(packed, qn2.reshape(N * R), kn2.reshape(N * R), xw2)
    return _make_final()(acc2, ext2, Wl, bl.reshape(1, OUT))


# speedup vs baseline: 58.7819x; 1.1020x over previous
import functools

import jax
import jax.numpy as jnp
from jax import lax
from jax.experimental import pallas as pl
from jax.experimental.pallas import tpu as pltpu
from jax.experimental.pallas import tpu_sc as plsc

N = 10000
E = 320000
IN = 128
H = 128
OUT = 128
R = 8
EW = 16
NC = 2
NS = 16
NW = NC * NS
PER_W = E // NW
B = 80
NB = PER_W // B
BN = 1000
ROWS_PER_SUB = 624
TAIL_ROW0 = NS * ROWS_PER_SUB
TAIL_ROWS = N - TAIL_ROW0


def _idx_body(src_ref, dst_ref, et_ref, p_ref):
    iq = dst_ref[...] * R + et_ref[...]
    p_ref[...] = jnp.bitwise_or(jnp.left_shift(iq, 14), src_ref[...])


def _make_idx():
    shp = (625, 512)
    spec = pl.BlockSpec(shp, lambda: (0, 0))
    return pl.pallas_call(
        _idx_body,
        grid=(),
        in_specs=[spec, spec, spec],
        out_specs=spec,
        out_shape=jax.ShapeDtypeStruct(shp, jnp.int32),
    )


def _dense_body(x_ref, w_ref, xw_ref):
    xw_ref[...] = jnp.dot(x_ref[...], w_ref[0],
                          preferred_element_type=jnp.float32)


def _make_dense():
    nblk = N // BN
    return pl.pallas_call(
        _dense_body,
        grid=(nblk, R),
        in_specs=[
            pl.BlockSpec((BN, IN), lambda nb, r: (nb, 0)),
            pl.BlockSpec((1, IN, H), lambda nb, r: (r, 0, 0)),
        ],
        out_specs=pl.BlockSpec((BN, H), lambda nb, r: (r * nblk + nb, 0)),
        out_shape=jax.ShapeDtypeStruct((R * N, H), jnp.float32),
    )


def _qk_body(x_ref, w_ref, q_ref, k_ref, qn_ref, kn_ref):
    wq = jnp.stack([jnp.dot(w_ref[r], q_ref[r, 0]) for r in range(R)], axis=0)
    wk = jnp.stack([jnp.dot(w_ref[r], k_ref[r, 0]) for r in range(R)], axis=0)
    dn = (((1,), (1,)), ((), ()))
    qn_ref[...] = lax.dot_general(x_ref[...], wq, dn,
                                  preferred_element_type=jnp.float32)
    kn_ref[...] = lax.dot_general(x_ref[...], wk, dn,
                                  preferred_element_type=jnp.float32)


def _make_qk():
    return pl.pallas_call(
        _qk_body,
        grid=(N // BN,),
        in_specs=[
            pl.BlockSpec((BN, IN), lambda nb: (nb, 0)),
            pl.BlockSpec((R, IN, H), lambda nb: (0, 0, 0)),
            pl.BlockSpec((R, 1, H), lambda nb: (0, 0, 0)),
            pl.BlockSpec((R, 1, H), lambda nb: (0, 0, 0)),
        ],
        out_specs=[
            pl.BlockSpec((BN, R), lambda nb: (nb, 0)),
            pl.BlockSpec((BN, R), lambda nb: (nb, 0)),
        ],
        out_shape=[jax.ShapeDtypeStruct((N, R), jnp.float32)] * 2,
    )


def _combine_body(acc_ref, ext_ref, h_ref):
    a = acc_ref[0] + acc_ref[1]
    den = ext_ref[0, :, 0] + ext_ref[1, :, 0]
    h_ref[...] = jnp.maximum(a / (den[:, None] + 1e-16), 0.0)


def _make_combine():
    return pl.pallas_call(
        _combine_body,
        grid=(N // BN,),
        in_specs=[
            pl.BlockSpec((2, BN, H), lambda nb: (0, nb, 0)),
            pl.BlockSpec((2, BN, EW), lambda nb: (0, nb, 0)),
        ],
        out_specs=pl.BlockSpec((BN, H), lambda nb: (nb, 0)),
        out_shape=jax.ShapeDtypeStruct((N, H), jnp.float32),
    )


def _final_body(acc_ref, ext_ref, wl_ref, bl_ref, y_ref):
    a = acc_ref[0] + acc_ref[1]
    den = ext_ref[0, :, 0] + ext_ref[1, :, 0]
    h = jnp.maximum(a / (den[:, None] + 1e-16), 0.0)
    y_ref[...] = jnp.dot(h, wl_ref[...],
                         preferred_element_type=jnp.float32) + bl_ref[0][None, :]


def _make_final():
    return pl.pallas_call(
        _final_body,
        grid=(N // BN,),
        in_specs=[
            pl.BlockSpec((2, BN, H), lambda nb: (0, nb, 0)),
            pl.BlockSpec((2, BN, EW), lambda nb: (0, nb, 0)),
            pl.BlockSpec((H, OUT), lambda nb: (0, 0)),
            pl.BlockSpec((1, OUT), lambda nb: (0, 0)),
        ],
        out_specs=pl.BlockSpec((BN, OUT), lambda nb: (nb, 0)),
        out_shape=jax.ShapeDtypeStruct((N, OUT), jnp.float32),
    )


def _edge_body(p_hbm, qn_hbm, kn_hbm, xw_hbm, acc_hbm, ext_hbm,
               packed, iqb, isb, ikb, dstb, qa, ka, wv, exb, rows2,
               acc_sh, ext_sh,
               sem_q0, sem_q1, sem_k0, sem_k1, sem_r0, sem_r1,
               sem_s0, sem_s1, sem_e0, sem_e1):
    c = lax.axis_index("c")
    s = lax.axis_index("s")
    wid = s * NC + c
    brow0 = wid * NB
    row0 = s * ROWS_PER_SUB

    for sl in range(2):
        def zb(b, cc):
            for j in range(H // 16):
                rows2[sl, b, pl.ds(j * 16, 16)] = jnp.zeros((16,), jnp.float32)
            return cc
        lax.fori_loop(0, B, zb, 0)
    for sl in range(2):
        def ze(b, cc):
            exb[sl, b, pl.ds(0, 16)] = jnp.zeros((16,), jnp.float32)
            return cc
        lax.fori_loop(0, B, ze, 0)
    for k in range(ROWS_PER_SUB // B):
        pltpu.sync_copy(rows2.at[0], acc_sh.at[pl.ds(row0 + k * B, B)])
        pltpu.sync_copy(exb.at[0], ext_sh.at[pl.ds(row0 + k * B, B)])
    rem = ROWS_PER_SUB % B
    pltpu.sync_copy(rows2.at[0].at[pl.ds(0, rem)],
                    acc_sh.at[pl.ds(row0 + (ROWS_PER_SUB // B) * B, rem)])
    pltpu.sync_copy(exb.at[0].at[pl.ds(0, rem)],
                    ext_sh.at[pl.ds(row0 + (ROWS_PER_SUB // B) * B, rem)])
    @pl.when(s == 0)
    def _zero_tail():
        pltpu.sync_copy(rows2.at[0].at[pl.ds(0, TAIL_ROWS)],
                        acc_sh.at[pl.ds(TAIL_ROW0, TAIL_ROWS)])
        pltpu.sync_copy(exb.at[0].at[pl.ds(0, TAIL_ROWS)],
                        ext_sh.at[pl.ds(TAIL_ROW0, TAIL_ROWS)])

    pltpu.sync_copy(p_hbm.at[pl.ds(brow0, NB)], packed)
    plsc.subcore_barrier()

    sem_q = (sem_q0, sem_q1)
    sem_k = (sem_k0, sem_k1)
    sem_r = (sem_r0, sem_r1)
    sem_s = (sem_s0, sem_s1)
    sem_e = (sem_e0, sem_e1)

    def fire(t, slot):
        @pl.when(t >= 2)
        def _drain():
            pltpu.make_async_copy(rows2.at[slot], acc_sh.at[dstb.at[slot]],
                                  sem_s[slot]).wait()
            pltpu.make_async_copy(exb.at[slot], ext_sh.at[dstb.at[slot]],
                                  sem_e[slot]).wait()
        for i in range(B // 16):
            sl16 = pl.ds(i * 16, 16)
            pch = packed[t, sl16]
            iqc = lax.shift_right_logical(pch, 14)
            srcv = jnp.bitwise_and(pch, 16383)
            etc = jnp.bitwise_and(iqc, R - 1)
            iqb[slot, sl16] = iqc
            dstb[slot, sl16] = lax.shift_right_logical(pch, 17)
            isb[slot, sl16] = srcv * R + etc
            ikb[slot, sl16] = etc * N + srcv
        pltpu.async_copy(qn_hbm.at[iqb.at[slot]], qa.at[slot], sem_q[slot])
        pltpu.async_copy(kn_hbm.at[isb.at[slot]], ka.at[slot], sem_k[slot])
        pltpu.async_copy(xw_hbm.at[ikb.at[slot]], rows2.at[slot], sem_r[slot])

    def process(t, slot):
        pltpu.make_async_copy(qn_hbm.at[iqb.at[slot]], qa.at[slot],
                              sem_q[slot]).wait()
        pltpu.make_async_copy(kn_hbm.at[isb.at[slot]], ka.at[slot],
                              sem_k[slot]).wait()
        pltpu.make_async_copy(xw_hbm.at[ikb.at[slot]], rows2.at[slot],
                              sem_r[slot]).wait()
        rr = rows2.at[slot]
        for i in range(B // 16):
            sl16 = pl.ds(i * 16, 16)
            bidx = lax.iota(jnp.int32, 16) + i * 16
            z = qa[slot, sl16] + ka[slot, sl16]
            z = jnp.maximum(z, 0.2 * z)
            w = jnp.exp(z)
            wv[sl16] = w
            plsc.store_scatter(exb.at[slot], [bidx, jnp.zeros((16,), jnp.int32)], w)
        @plsc.parallel_loop(0, B, 1, unroll=8)
        def mulb(b):
            wb = plsc.load_gather(wv, [jnp.full((16,), 0, jnp.int32) + b])
            for j in range(H // 16):
                rr[b, pl.ds(j * 16, 16)] = rr[b, pl.ds(j * 16, 16)] * wb
        pltpu.async_copy(rr, acc_sh.at[dstb.at[slot]], sem_s[slot], add=True)
        pltpu.async_copy(exb.at[slot], ext_sh.at[dstb.at[slot]], sem_e[slot],
                         add=True)

    fire(0, 0)
    def body(u, carry):
        t0 = 2 * u
        @pl.when(t0 + 1 < NB)
        def _f1():
            fire(t0 + 1, 1)
        process(t0, 0)
        @pl.when(t0 + 2 < NB)
        def _f0():
            fire(t0 + 2, 0)
        @pl.when(t0 + 1 < NB)
        def _p1():
            process(t0 + 1, 1)
        return carry
    lax.fori_loop(0, (NB + 1) // 2, body, 0)
    for slot in range(2):
        pltpu.make_async_copy(rows2.at[slot], acc_sh.at[dstb.at[slot]],
                              sem_s[slot]).wait()
        pltpu.make_async_copy(exb.at[slot], ext_sh.at[dstb.at[slot]],
                              sem_e[slot]).wait()
    plsc.subcore_barrier()

    pltpu.sync_copy(acc_sh.at[pl.ds(row0, ROWS_PER_SUB)],
                    acc_hbm.at[c, pl.ds(row0, ROWS_PER_SUB)])
    pltpu.sync_copy(ext_sh.at[pl.ds(row0, ROWS_PER_SUB)],
                    ext_hbm.at[c, pl.ds(row0, ROWS_PER_SUB)])
    @pl.when(s == 0)
    def _dump_tail():
        pltpu.sync_copy(acc_sh.at[pl.ds(TAIL_ROW0, TAIL_ROWS)],
                        acc_hbm.at[c, pl.ds(TAIL_ROW0, TAIL_ROWS)])
        pltpu.sync_copy(ext_sh.at[pl.ds(TAIL_ROW0, TAIL_ROWS)],
                        ext_hbm.at[c, pl.ds(TAIL_ROW0, TAIL_ROWS)])


def _make_edge():
    mesh = plsc.VectorSubcoreMesh(core_axis_name="c", subcore_axis_name="s")
    return functools.partial(
        pl.kernel,
        out_type=[
            jax.ShapeDtypeStruct((NC, N, H), jnp.float32),
            jax.ShapeDtypeStruct((NC, N, EW), jnp.float32),
        ],
        mesh=mesh,
        compiler_params=pltpu.CompilerParams(use_tc_tiling_on_sc=False,
                                             needs_layout_passes=False),
        scratch_types=[
            pltpu.VMEM((NB, B), jnp.int32),
            pltpu.VMEM((2, B), jnp.int32),
            pltpu.VMEM((2, B), jnp.int32),
            pltpu.VMEM((2, B), jnp.int32),
            pltpu.VMEM((2, B), jnp.int32),
            pltpu.VMEM((2, B), jnp.float32),
            pltpu.VMEM((2, B), jnp.float32),
            pltpu.VMEM((B,), jnp.float32),
            pltpu.VMEM((2, B, EW), jnp.float32),
            pltpu.VMEM((2, B, H), jnp.float32),
            pltpu.VMEM_SHARED((N, H), jnp.float32),
            pltpu.VMEM_SHARED((N, EW), jnp.float32),
        ] + [pltpu.SemaphoreType.DMA] * 10,
    )(_edge_body)


def kernel(x, edge_index, edge_type, W1, Q1, K1, W2, Q2, K2, Wl, bl):
    src = edge_index[0].reshape(625, 512)
    dst = edge_index[1].reshape(625, 512)
    et = edge_type.reshape(625, 512)
    packed = _make_idx()(src, dst, et).reshape(E // B, B)

    dense = _make_dense()
    qk = _make_qk()
    edge = _make_edge()
    combine = _make_combine()

    xw1 = dense(x, W1)
    qn1, kn1 = qk(x, W1, Q1.reshape(R, 1, H), K1.reshape(R, 1, H))
    acc1, ext1 = edge(packed, qn1.reshape(N * R), kn1.reshape(N * R), xw1)
    h1 = combine(acc1, ext1)

    xw2 = dense(h1, W2)
    qn2, kn2 = qk(h1, W2, Q2.reshape(R, 1, H), K2.reshape(R, 1, H))
    acc2, ext2 = edge(packed, qn2.reshape(N * R), kn2.reshape(N * R), xw2)
    return _make_final()(acc2, ext2, Wl, bl.reshape(1, OUT))


# speedup vs baseline: 65.3321x; 1.1114x over previous
import functools

import jax
import jax.numpy as jnp
from jax import lax
from jax.experimental import pallas as pl
from jax.experimental.pallas import tpu as pltpu
from jax.experimental.pallas import tpu_sc as plsc

N = 10000
E = 320000
IN = 128
H = 128
OUT = 128
R = 8
EW = 16
NC = 2
NS = 16
NW = NC * NS
PER_W = E // NW
B = 80
NB = PER_W // B
BN = 1000
ROWS_PER_SUB = 624
TAIL_ROW0 = NS * ROWS_PER_SUB
TAIL_ROWS = N - TAIL_ROW0


def _idx_body(src_ref, dst_ref, et_ref, p_ref):
    iq = dst_ref[...] * R + et_ref[...]
    p_ref[...] = jnp.bitwise_or(jnp.left_shift(iq, 14), src_ref[...])


def _make_idx():
    shp = (625, 512)
    spec = pl.BlockSpec(shp, lambda: (0, 0))
    return pl.pallas_call(
        _idx_body,
        grid=(),
        in_specs=[spec, spec, spec],
        out_specs=spec,
        out_shape=jax.ShapeDtypeStruct(shp, jnp.int32),
    )


DBN = 2000


def _dense_body(x_ref, w_ref, q_ref, k_ref, xw_ref, qn_ref, kn_ref):
    r = pl.program_id(1)
    xw = jnp.dot(x_ref[...], w_ref[0], preferred_element_type=jnp.float32)
    xw_ref[...] = xw
    col = lax.broadcasted_iota(jnp.int32, (DBN, R), 1)
    qv = jnp.dot(xw, q_ref[0, 0], preferred_element_type=jnp.float32)
    kv = jnp.dot(xw, k_ref[0, 0], preferred_element_type=jnp.float32)
    qn_ref[...] = jnp.where(col == r, qv[:, None], qn_ref[...])
    kn_ref[...] = jnp.where(col == r, kv[:, None], kn_ref[...])


def _make_dense():
    nblk = N // DBN
    return pl.pallas_call(
        _dense_body,
        grid=(nblk, R),
        in_specs=[
            pl.BlockSpec((DBN, IN), lambda nb, r: (nb, 0)),
            pl.BlockSpec((1, IN, H), lambda nb, r: (r, 0, 0)),
            pl.BlockSpec((1, 1, H), lambda nb, r: (r, 0, 0)),
            pl.BlockSpec((1, 1, H), lambda nb, r: (r, 0, 0)),
        ],
        out_specs=[
            pl.BlockSpec((DBN, H), lambda nb, r: (r * nblk + nb, 0)),
            pl.BlockSpec((DBN, R), lambda nb, r: (nb, 0)),
            pl.BlockSpec((DBN, R), lambda nb, r: (nb, 0)),
        ],
        out_shape=[
            jax.ShapeDtypeStruct((R * N, H), jnp.float32),
            jax.ShapeDtypeStruct((N, R), jnp.float32),
            jax.ShapeDtypeStruct((N, R), jnp.float32),
        ],
    )


def _combine_body(acc_ref, ext_ref, h_ref):
    a = acc_ref[0] + acc_ref[1]
    den = ext_ref[0, :, 0] + ext_ref[1, :, 0]
    h_ref[...] = jnp.maximum(a / (den[:, None] + 1e-16), 0.0)


def _make_combine():
    return pl.pallas_call(
        _combine_body,
        grid=(N // BN,),
        in_specs=[
            pl.BlockSpec((2, BN, H), lambda nb: (0, nb, 0)),
            pl.BlockSpec((2, BN, EW), lambda nb: (0, nb, 0)),
        ],
        out_specs=pl.BlockSpec((BN, H), lambda nb: (nb, 0)),
        out_shape=jax.ShapeDtypeStruct((N, H), jnp.float32),
    )


def _final_body(acc_ref, ext_ref, wl_ref, bl_ref, y_ref):
    a = acc_ref[0] + acc_ref[1]
    den = ext_ref[0, :, 0] + ext_ref[1, :, 0]
    h = jnp.maximum(a / (den[:, None] + 1e-16), 0.0)
    y_ref[...] = jnp.dot(h, wl_ref[...],
                         preferred_element_type=jnp.float32) + bl_ref[0][None, :]


def _make_final():
    return pl.pallas_call(
        _final_body,
        grid=(N // BN,),
        in_specs=[
            pl.BlockSpec((2, BN, H), lambda nb: (0, nb, 0)),
            pl.BlockSpec((2, BN, EW), lambda nb: (0, nb, 0)),
            pl.BlockSpec((H, OUT), lambda nb: (0, 0)),
            pl.BlockSpec((1, OUT), lambda nb: (0, 0)),
        ],
        out_specs=pl.BlockSpec((BN, OUT), lambda nb: (nb, 0)),
        out_shape=jax.ShapeDtypeStruct((N, OUT), jnp.float32),
    )


def _edge_body(p_hbm, qn_hbm, kn_hbm, xw_hbm, acc_hbm, ext_hbm,
               packed, iqb, isb, ikb, dstb, qa, ka, wv, exb, rows2,
               acc_sh, ext_sh,
               sem_q0, sem_q1, sem_k0, sem_k1, sem_r0, sem_r1,
               sem_s0, sem_s1, sem_e0, sem_e1):
    c = lax.axis_index("c")
    s = lax.axis_index("s")
    wid = s * NC + c
    brow0 = wid * NB
    row0 = s * ROWS_PER_SUB

    for sl in range(2):
        def zb(b, cc):
            for j in range(H // 16):
                rows2[sl, b, pl.ds(j * 16, 16)] = jnp.zeros((16,), jnp.float32)
            return cc
        lax.fori_loop(0, B, zb, 0)
    for sl in range(2):
        def ze(b, cc):
            exb[sl, b, pl.ds(0, 16)] = jnp.zeros((16,), jnp.float32)
            return cc
        lax.fori_loop(0, B, ze, 0)
    for k in range(ROWS_PER_SUB // B):
        pltpu.sync_copy(rows2.at[0], acc_sh.at[pl.ds(row0 + k * B, B)])
        pltpu.sync_copy(exb.at[0], ext_sh.at[pl.ds(row0 + k * B, B)])
    rem = ROWS_PER_SUB % B
    pltpu.sync_copy(rows2.at[0].at[pl.ds(0, rem)],
                    acc_sh.at[pl.ds(row0 + (ROWS_PER_SUB // B) * B, rem)])
    pltpu.sync_copy(exb.at[0].at[pl.ds(0, rem)],
                    ext_sh.at[pl.ds(row0 + (ROWS_PER_SUB // B) * B, rem)])
    @pl.when(s == 0)
    def _zero_tail():
        pltpu.sync_copy(rows2.at[0].at[pl.ds(0, TAIL_ROWS)],
                        acc_sh.at[pl.ds(TAIL_ROW0, TAIL_ROWS)])
        pltpu.sync_copy(exb.at[0].at[pl.ds(0, TAIL_ROWS)],
                        ext_sh.at[pl.ds(TAIL_ROW0, TAIL_ROWS)])

    pltpu.sync_copy(p_hbm.at[pl.ds(brow0, NB)], packed)
    plsc.subcore_barrier()

    sem_q = (sem_q0, sem_q1)
    sem_k = (sem_k0, sem_k1)
    sem_r = (sem_r0, sem_r1)
    sem_s = (sem_s0, sem_s1)
    sem_e = (sem_e0, sem_e1)

    def fire(t, slot):
        @pl.when(t >= 2)
        def _drain():
            pltpu.make_async_copy(rows2.at[slot], acc_sh.at[dstb.at[slot]],
                                  sem_s[slot]).wait()
            pltpu.make_async_copy(exb.at[slot], ext_sh.at[dstb.at[slot]],
                                  sem_e[slot]).wait()
        for i in range(B // 16):
            sl16 = pl.ds(i * 16, 16)
            pch = packed[t, sl16]
            iqc = lax.shift_right_logical(pch, 14)
            srcv = jnp.bitwise_and(pch, 16383)
            etc = jnp.bitwise_and(iqc, R - 1)
            iqb[slot, sl16] = iqc
            dstb[slot, sl16] = lax.shift_right_logical(pch, 17)
            isb[slot, sl16] = srcv * R + etc
            ikb[slot, sl16] = etc * N + srcv
        pltpu.async_copy(qn_hbm.at[iqb.at[slot]], qa.at[slot], sem_q[slot])
        pltpu.async_copy(kn_hbm.at[isb.at[slot]], ka.at[slot], sem_k[slot])
        pltpu.async_copy(xw_hbm.at[ikb.at[slot]], rows2.at[slot], sem_r[slot])

    def process(t, slot):
        pltpu.make_async_copy(qn_hbm.at[iqb.at[slot]], qa.at[slot],
                              sem_q[slot]).wait()
        pltpu.make_async_copy(kn_hbm.at[isb.at[slot]], ka.at[slot],
                              sem_k[slot]).wait()
        pltpu.make_async_copy(xw_hbm.at[ikb.at[slot]], rows2.at[slot],
                              sem_r[slot]).wait()
        rr = rows2.at[slot]
        for i in range(B // 16):
            sl16 = pl.ds(i * 16, 16)
            bidx = lax.iota(jnp.int32, 16) + i * 16
            z = qa[slot, sl16] + ka[slot, sl16]
            z = jnp.maximum(z, 0.2 * z)
            w = jnp.exp(z)
            wv[sl16] = w
            plsc.store_scatter(exb.at[slot], [bidx, jnp.zeros((16,), jnp.int32)], w)
        @plsc.parallel_loop(0, B, 1, unroll=8)
        def mulb(b):
            wb = plsc.load_gather(wv, [jnp.full((16,), 0, jnp.int32) + b])
            for j in range(H // 16):
                rr[b, pl.ds(j * 16, 16)] = rr[b, pl.ds(j * 16, 16)] * wb
        pltpu.async_copy(rr, acc_sh.at[dstb.at[slot]], sem_s[slot], add=True)
        pltpu.async_copy(exb.at[slot], ext_sh.at[dstb.at[slot]], sem_e[slot],
                         add=True)

    fire(0, 0)
    def body(u, carry):
        t0 = 2 * u
        @pl.when(t0 + 1 < NB)
        def _f1():
            fire(t0 + 1, 1)
        process(t0, 0)
        @pl.when(t0 + 2 < NB)
        def _f0():
            fire(t0 + 2, 0)
        @pl.when(t0 + 1 < NB)
        def _p1():
            process(t0 + 1, 1)
        return carry
    lax.fori_loop(0, (NB + 1) // 2, body, 0)
    for slot in range(2):
        pltpu.make_async_copy(rows2.at[slot], acc_sh.at[dstb.at[slot]],
                              sem_s[slot]).wait()
        pltpu.make_async_copy(exb.at[slot], ext_sh.at[dstb.at[slot]],
                              sem_e[slot]).wait()
    plsc.subcore_barrier()

    pltpu.sync_copy(acc_sh.at[pl.ds(row0, ROWS_PER_SUB)],
                    acc_hbm.at[c, pl.ds(row0, ROWS_PER_SUB)])
    pltpu.sync_copy(ext_sh.at[pl.ds(row0, ROWS_PER_SUB)],
                    ext_hbm.at[c, pl.ds(row0, ROWS_PER_SUB)])
    @pl.when(s == 0)
    def _dump_tail():
        pltpu.sync_copy(acc_sh.at[pl.ds(TAIL_ROW0, TAIL_ROWS)],
                        acc_hbm.at[c, pl.ds(TAIL_ROW0, TAIL_ROWS)])
        pltpu.sync_copy(ext_sh.at[pl.ds(TAIL_ROW0, TAIL_ROWS)],
                        ext_hbm.at[c, pl.ds(TAIL_ROW0, TAIL_ROWS)])


def _make_edge():
    mesh = plsc.VectorSubcoreMesh(core_axis_name="c", subcore_axis_name="s")
    return functools.partial(
        pl.kernel,
        out_type=[
            jax.ShapeDtypeStruct((NC, N, H), jnp.float32),
            jax.ShapeDtypeStruct((NC, N, EW), jnp.float32),
        ],
        mesh=mesh,
        compiler_params=pltpu.CompilerParams(use_tc_tiling_on_sc=False,
                                             needs_layout_passes=False),
        scratch_types=[
            pltpu.VMEM((NB, B), jnp.int32),
            pltpu.VMEM((2, B), jnp.int32),
            pltpu.VMEM((2, B), jnp.int32),
            pltpu.VMEM((2, B), jnp.int32),
            pltpu.VMEM((2, B), jnp.int32),
            pltpu.VMEM((2, B), jnp.float32),
            pltpu.VMEM((2, B), jnp.float32),
            pltpu.VMEM((B,), jnp.float32),
            pltpu.VMEM((2, B, EW), jnp.float32),
            pltpu.VMEM((2, B, H), jnp.float32),
            pltpu.VMEM_SHARED((N, H), jnp.float32),
            pltpu.VMEM_SHARED((N, EW), jnp.float32),
        ] + [pltpu.SemaphoreType.DMA] * 10,
    )(_edge_body)


def kernel(x, edge_index, edge_type, W1, Q1, K1, W2, Q2, K2, Wl, bl):
    src = edge_index[0].reshape(625, 512)
    dst = edge_index[1].reshape(625, 512)
    et = edge_type.reshape(625, 512)
    packed = _make_idx()(src, dst, et).reshape(E // B, B)

    dense = _make_dense()
    edge = _make_edge()
    combine = _make_combine()

    xw1, qn1, kn1 = dense(x, W1, Q1.reshape(R, 1, H), K1.reshape(R, 1, H))
    acc1, ext1 = edge(packed, qn1.reshape(N * R), kn1.reshape(N * R), xw1)
    h1 = combine(acc1, ext1)

    xw2, qn2, kn2 = dense(h1, W2, Q2.reshape(R, 1, H), K2.reshape(R, 1, H))
    acc2, ext2 = edge(packed, qn2.reshape(N * R), kn2.reshape(N * R), xw2)
    return _make_final()(acc2, ext2, Wl, bl.reshape(1, OUT))


# speedup vs baseline: 66.2879x; 1.0146x over previous
import functools

import jax
import jax.numpy as jnp
from jax import lax
from jax.experimental import pallas as pl
from jax.experimental.pallas import tpu as pltpu
from jax.experimental.pallas import tpu_sc as plsc

N = 10000
E = 320000
IN = 128
H = 128
OUT = 128
R = 8
EW = 16
NC = 2
NS = 16
NW = NC * NS
PER_W = E // NW
B = 80
NB = PER_W // B
BN = 1000
ROWS_PER_SUB = 624
TAIL_ROW0 = NS * ROWS_PER_SUB
TAIL_ROWS = N - TAIL_ROW0


def _idx_body(src_ref, dst_ref, et_ref, p_ref):
    iq = dst_ref[...] * R + et_ref[...]
    p_ref[...] = jnp.bitwise_or(jnp.left_shift(iq, 14), src_ref[...])


def _make_idx():
    shp = (625, 512)
    spec = pl.BlockSpec(shp, lambda: (0, 0))
    return pl.pallas_call(
        _idx_body,
        grid=(),
        in_specs=[spec, spec, spec],
        out_specs=spec,
        out_shape=jax.ShapeDtypeStruct(shp, jnp.int32),
    )


DBN = 2000


def _dense_body(x_ref, w_ref, q_ref, k_ref, xw_ref, qn_ref, kn_ref):
    r = pl.program_id(1)
    xw = jnp.dot(x_ref[...], w_ref[0], preferred_element_type=jnp.float32)
    xw_ref[...] = xw
    col = lax.broadcasted_iota(jnp.int32, (DBN, R), 1)
    qv = jnp.dot(xw, q_ref[0, 0], preferred_element_type=jnp.float32)
    kv = jnp.dot(xw, k_ref[0, 0], preferred_element_type=jnp.float32)
    qn_ref[...] = jnp.where(col == r, qv[:, None], qn_ref[...])
    kn_ref[...] = jnp.where(col == r, kv[:, None], kn_ref[...])


def _make_dense():
    nblk = N // DBN
    return pl.pallas_call(
        _dense_body,
        grid=(nblk, R),
        in_specs=[
            pl.BlockSpec((DBN, IN), lambda nb, r: (nb, 0)),
            pl.BlockSpec((1, IN, H), lambda nb, r: (r, 0, 0)),
            pl.BlockSpec((1, 1, H), lambda nb, r: (r, 0, 0)),
            pl.BlockSpec((1, 1, H), lambda nb, r: (r, 0, 0)),
        ],
        out_specs=[
            pl.BlockSpec((DBN, H), lambda nb, r: (r * nblk + nb, 0)),
            pl.BlockSpec((DBN, R), lambda nb, r: (nb, 0)),
            pl.BlockSpec((DBN, R), lambda nb, r: (nb, 0)),
        ],
        out_shape=[
            jax.ShapeDtypeStruct((R * N, H), jnp.float32),
            jax.ShapeDtypeStruct((N, R), jnp.float32),
            jax.ShapeDtypeStruct((N, R), jnp.float32),
        ],
    )


def _combine_body(acc_ref, ext_ref, h_ref):
    a = acc_ref[0] + acc_ref[1]
    den = ext_ref[0, :, 0] + ext_ref[1, :, 0]
    h_ref[...] = jnp.maximum(a / (den[:, None] + 1e-16), 0.0)


def _make_combine():
    return pl.pallas_call(
        _combine_body,
        grid=(N // BN,),
        in_specs=[
            pl.BlockSpec((2, BN, H), lambda nb: (0, nb, 0)),
            pl.BlockSpec((2, BN, EW), lambda nb: (0, nb, 0)),
        ],
        out_specs=pl.BlockSpec((BN, H), lambda nb: (nb, 0)),
        out_shape=jax.ShapeDtypeStruct((N, H), jnp.float32),
    )


def _final_body(acc_ref, ext_ref, wl_ref, bl_ref, y_ref):
    a = acc_ref[0] + acc_ref[1]
    den = ext_ref[0, :, 0] + ext_ref[1, :, 0]
    h = jnp.maximum(a / (den[:, None] + 1e-16), 0.0)
    y_ref[...] = jnp.dot(h, wl_ref[...],
                         preferred_element_type=jnp.float32) + bl_ref[0][None, :]


def _make_final():
    return pl.pallas_call(
        _final_body,
        grid=(N // BN,),
        in_specs=[
            pl.BlockSpec((2, BN, H), lambda nb: (0, nb, 0)),
            pl.BlockSpec((2, BN, EW), lambda nb: (0, nb, 0)),
            pl.BlockSpec((H, OUT), lambda nb: (0, 0)),
            pl.BlockSpec((1, OUT), lambda nb: (0, 0)),
        ],
        out_specs=pl.BlockSpec((BN, OUT), lambda nb: (nb, 0)),
        out_shape=jax.ShapeDtypeStruct((N, OUT), jnp.float32),
    )


def _edge_body(p_hbm, qn_hbm, kn_hbm, xw_hbm, acc_hbm, ext_hbm,
               packed, iqb, isb, ikb, dstb, qa, ka, wv, exb, rows2,
               acc_sh, ext_sh,
               sem_q0, sem_q1, sem_k0, sem_k1, sem_r0, sem_r1,
               sem_s0, sem_s1, sem_e0, sem_e1):
    c = lax.axis_index("c")
    s = lax.axis_index("s")
    wid = s * NC + c
    brow0 = wid * NB
    row0 = s * ROWS_PER_SUB

    for sl in range(2):
        def zb(b, cc):
            for j in range(H // 16):
                rows2[sl, b, pl.ds(j * 16, 16)] = jnp.zeros((16,), jnp.float32)
            return cc
        lax.fori_loop(0, B, zb, 0)
    for sl in range(2):
        def ze(b, cc):
            exb[sl, b, pl.ds(0, 16)] = jnp.zeros((16,), jnp.float32)
            return cc
        lax.fori_loop(0, B, ze, 0)
    for k in range(ROWS_PER_SUB // B):
        pltpu.sync_copy(rows2.at[0], acc_sh.at[pl.ds(row0 + k * B, B)])
        pltpu.sync_copy(exb.at[0], ext_sh.at[pl.ds(row0 + k * B, B)])
    rem = ROWS_PER_SUB % B
    pltpu.sync_copy(rows2.at[0].at[pl.ds(0, rem)],
                    acc_sh.at[pl.ds(row0 + (ROWS_PER_SUB // B) * B, rem)])
    pltpu.sync_copy(exb.at[0].at[pl.ds(0, rem)],
                    ext_sh.at[pl.ds(row0 + (ROWS_PER_SUB // B) * B, rem)])
    @pl.when(s == 0)
    def _zero_tail():
        pltpu.sync_copy(rows2.at[0].at[pl.ds(0, TAIL_ROWS)],
                        acc_sh.at[pl.ds(TAIL_ROW0, TAIL_ROWS)])
        pltpu.sync_copy(exb.at[0].at[pl.ds(0, TAIL_ROWS)],
                        ext_sh.at[pl.ds(TAIL_ROW0, TAIL_ROWS)])

    pltpu.sync_copy(p_hbm.at[pl.ds(brow0, NB)], packed)
    plsc.subcore_barrier()

    sem_q = (sem_q0, sem_q1)
    sem_k = (sem_k0, sem_k1)
    sem_r = (sem_r0, sem_r1)
    sem_s = (sem_s0, sem_s1)
    sem_e = (sem_e0, sem_e1)

    def fire(t, slot):
        @pl.when(t >= 2)
        def _drain():
            pltpu.make_async_copy(rows2.at[slot], acc_sh.at[dstb.at[slot]],
                                  sem_s[slot]).wait()
            pltpu.make_async_copy(exb.at[slot], ext_sh.at[dstb.at[slot]],
                                  sem_e[slot]).wait()
        for i in range(B // 16):
            sl16 = pl.ds(i * 16, 16)
            pch = packed[t, sl16]
            iqc = lax.shift_right_logical(pch, 14)
            srcv = jnp.bitwise_and(pch, 16383)
            etc = jnp.bitwise_and(iqc, R - 1)
            iqb[slot, sl16] = iqc
            dstb[slot, sl16] = lax.shift_right_logical(pch, 17)
            isb[slot, sl16] = srcv * R + etc
            ikb[slot, sl16] = etc * N + srcv
        pltpu.async_copy(qn_hbm.at[iqb.at[slot]], qa.at[slot], sem_q[slot])
        pltpu.async_copy(kn_hbm.at[isb.at[slot]], ka.at[slot], sem_k[slot])
        pltpu.async_copy(xw_hbm.at[ikb.at[slot]], rows2.at[slot], sem_r[slot])

    def process(t, slot):
        pltpu.make_async_copy(qn_hbm.at[iqb.at[slot]], qa.at[slot],
                              sem_q[slot]).wait()
        pltpu.make_async_copy(kn_hbm.at[isb.at[slot]], ka.at[slot],
                              sem_k[slot]).wait()
        rr = rows2.at[slot]
        for i in range(B // 16):
            sl16 = pl.ds(i * 16, 16)
            bidx = lax.iota(jnp.int32, 16) + i * 16
            z = qa[slot, sl16] + ka[slot, sl16]
            z = jnp.maximum(z, 0.2 * z)
            w = jnp.exp(z)
            wv[sl16] = w
            plsc.store_scatter(exb.at[slot], [bidx, jnp.zeros((16,), jnp.int32)], w)
        pltpu.make_async_copy(xw_hbm.at[ikb.at[slot]], rows2.at[slot],
                              sem_r[slot]).wait()
        @plsc.parallel_loop(0, B, 1, unroll=16)
        def mulb(b):
            wb = plsc.load_gather(wv, [jnp.full((16,), 0, jnp.int32) + b])
            for j in range(H // 16):
                rr[b, pl.ds(j * 16, 16)] = rr[b, pl.ds(j * 16, 16)] * wb
        pltpu.async_copy(rr, acc_sh.at[dstb.at[slot]], sem_s[slot], add=True)
        pltpu.async_copy(exb.at[slot], ext_sh.at[dstb.at[slot]], sem_e[slot],
                         add=True)

    fire(0, 0)
    def body(u, carry):
        t0 = 2 * u
        @pl.when(t0 + 1 < NB)
        def _f1():
            fire(t0 + 1, 1)
        process(t0, 0)
        @pl.when(t0 + 2 < NB)
        def _f0():
            fire(t0 + 2, 0)
        @pl.when(t0 + 1 < NB)
        def _p1():
            process(t0 + 1, 1)
        return carry
    lax.fori_loop(0, (NB + 1) // 2, body, 0)
    for slot in range(2):
        pltpu.make_async_copy(rows2.at[slot], acc_sh.at[dstb.at[slot]],
                              sem_s[slot]).wait()
        pltpu.make_async_copy(exb.at[slot], ext_sh.at[dstb.at[slot]],
                              sem_e[slot]).wait()
    plsc.subcore_barrier()

    pltpu.sync_copy(acc_sh.at[pl.ds(row0, ROWS_PER_SUB)],
                    acc_hbm.at[c, pl.ds(row0, ROWS_PER_SUB)])
    pltpu.sync_copy(ext_sh.at[pl.ds(row0, ROWS_PER_SUB)],
                    ext_hbm.at[c, pl.ds(row0, ROWS_PER_SUB)])
    @pl.when(s == 0)
    def _dump_tail():
        pltpu.sync_copy(acc_sh.at[pl.ds(TAIL_ROW0, TAIL_ROWS)],
                        acc_hbm.at[c, pl.ds(TAIL_ROW0, TAIL_ROWS)])
        pltpu.sync_copy(ext_sh.at[pl.ds(TAIL_ROW0, TAIL_ROWS)],
                        ext_hbm.at[c, pl.ds(TAIL_ROW0, TAIL_ROWS)])


def _make_edge():
    mesh = plsc.VectorSubcoreMesh(core_axis_name="c", subcore_axis_name="s")
    return functools.partial(
        pl.kernel,
        out_type=[
            jax.ShapeDtypeStruct((NC, N, H), jnp.float32),
            jax.ShapeDtypeStruct((NC, N, EW), jnp.float32),
        ],
        mesh=mesh,
        compiler_params=pltpu.CompilerParams(use_tc_tiling_on_sc=False,
                                             needs_layout_passes=False),
        scratch_types=[
            pltpu.VMEM((NB, B), jnp.int32),
            pltpu.VMEM((2, B), jnp.int32),
            pltpu.VMEM((2, B), jnp.int32),
            pltpu.VMEM((2, B), jnp.int32),
            pltpu.VMEM((2, B), jnp.int32),
            pltpu.VMEM((2, B), jnp.float32),
            pltpu.VMEM((2, B), jnp.float32),
            pltpu.VMEM((B,), jnp.float32),
            pltpu.VMEM((2, B, EW), jnp.float32),
            pltpu.VMEM((2, B, H), jnp.float32),
            pltpu.VMEM_SHARED((N, H), jnp.float32),
            pltpu.VMEM_SHARED((N, EW), jnp.float32),
        ] + [pltpu.SemaphoreType.DMA] * 10,
    )(_edge_body)


def kernel(x, edge_index, edge_type, W1, Q1, K1, W2, Q2, K2, Wl, bl):
    src = edge_index[0].reshape(625, 512)
    dst = edge_index[1].reshape(625, 512)
    et = edge_type.reshape(625, 512)
    packed = _make_idx()(src, dst, et).reshape(E // B, B)

    dense = _make_dense()
    edge = _make_edge()
    combine = _make_combine()

    xw1, qn1, kn1 = dense(x, W1, Q1.reshape(R, 1, H), K1.reshape(R, 1, H))
    acc1, ext1 = edge(packed, qn1.reshape(N * R), kn1.reshape(N * R), xw1)
    h1 = combine(acc1, ext1)

    xw2, qn2, kn2 = dense(h1, W2, Q2.reshape(R, 1, H), K2.reshape(R, 1, H))
    acc2, ext2 = edge(packed, qn2.reshape(N * R), kn2.reshape(N * R), xw2)
    return _make_final()(acc2, ext2, Wl, bl.reshape(1, OUT))


# speedup vs baseline: 69.9874x; 1.0558x over previous
import functools

import jax
import jax.numpy as jnp
from jax import lax
from jax.experimental import pallas as pl
from jax.experimental.pallas import tpu as pltpu
from jax.experimental.pallas import tpu_sc as plsc

N = 10000
E = 320000
IN = 128
H = 128
OUT = 128
R = 8
EW = 16
NC = 2
NS = 16
NW = NC * NS
PER_W = E // NW
B = 80
NB = PER_W // B
BN = 1000
ROWS_PER_SUB = 624
TAIL_ROW0 = NS * ROWS_PER_SUB
TAIL_ROWS = N - TAIL_ROW0


def _idx_body(src_ref, dst_ref, et_ref, p_ref):
    iq = dst_ref[...] * R + et_ref[...]
    p_ref[...] = jnp.bitwise_or(jnp.left_shift(iq, 14), src_ref[...])


def _make_idx():
    shp = (625, 512)
    spec = pl.BlockSpec(shp, lambda: (0, 0))
    return pl.pallas_call(
        _idx_body,
        grid=(),
        in_specs=[spec, spec, spec],
        out_specs=spec,
        out_shape=jax.ShapeDtypeStruct(shp, jnp.int32),
    )


DBN = 5000


def _dense_body(x_ref, w_ref, q_ref, k_ref, xw_ref, qn_ref, kn_ref):
    r = pl.program_id(1)
    xw = jnp.dot(x_ref[...], w_ref[0], preferred_element_type=jnp.float32)
    xw_ref[...] = xw
    col = lax.broadcasted_iota(jnp.int32, (DBN, R), 1)
    qv = jnp.dot(xw, q_ref[0, 0], preferred_element_type=jnp.float32)
    kv = jnp.dot(xw, k_ref[0, 0], preferred_element_type=jnp.float32)
    qn_ref[...] = jnp.where(col == r, qv[:, None], qn_ref[...])
    kn_ref[...] = jnp.where(col == r, kv[:, None], kn_ref[...])


def _make_dense():
    nblk = N // DBN
    return pl.pallas_call(
        _dense_body,
        grid=(nblk, R),
        in_specs=[
            pl.BlockSpec((DBN, IN), lambda nb, r: (nb, 0)),
            pl.BlockSpec((1, IN, H), lambda nb, r: (r, 0, 0)),
            pl.BlockSpec((1, 1, H), lambda nb, r: (r, 0, 0)),
            pl.BlockSpec((1, 1, H), lambda nb, r: (r, 0, 0)),
        ],
        out_specs=[
            pl.BlockSpec((DBN, H), lambda nb, r: (r * nblk + nb, 0)),
            pl.BlockSpec((DBN, R), lambda nb, r: (nb, 0)),
            pl.BlockSpec((DBN, R), lambda nb, r: (nb, 0)),
        ],
        out_shape=[
            jax.ShapeDtypeStruct((R * N, H), jnp.float32),
            jax.ShapeDtypeStruct((N, R), jnp.float32),
            jax.ShapeDtypeStruct((N, R), jnp.float32),
        ],
    )


def _combine_body(acc_ref, ext_ref, h_ref):
    a = acc_ref[0] + acc_ref[1]
    den = ext_ref[0, :, 0] + ext_ref[1, :, 0]
    h_ref[...] = jnp.maximum(a / (den[:, None] + 1e-16), 0.0)


def _make_combine():
    return pl.pallas_call(
        _combine_body,
        grid=(N // BN,),
        in_specs=[
            pl.BlockSpec((2, BN, H), lambda nb: (0, nb, 0)),
            pl.BlockSpec((2, BN, EW), lambda nb: (0, nb, 0)),
        ],
        out_specs=pl.BlockSpec((BN, H), lambda nb: (nb, 0)),
        out_shape=jax.ShapeDtypeStruct((N, H), jnp.float32),
    )


def _final_body(acc_ref, ext_ref, wl_ref, bl_ref, y_ref):
    a = acc_ref[0] + acc_ref[1]
    den = ext_ref[0, :, 0] + ext_ref[1, :, 0]
    h = jnp.maximum(a / (den[:, None] + 1e-16), 0.0)
    y_ref[...] = jnp.dot(h, wl_ref[...],
                         preferred_element_type=jnp.float32) + bl_ref[0][None, :]


def _make_final():
    return pl.pallas_call(
        _final_body,
        grid=(N // BN,),
        in_specs=[
            pl.BlockSpec((2, BN, H), lambda nb: (0, nb, 0)),
            pl.BlockSpec((2, BN, EW), lambda nb: (0, nb, 0)),
            pl.BlockSpec((H, OUT), lambda nb: (0, 0)),
            pl.BlockSpec((1, OUT), lambda nb: (0, 0)),
        ],
        out_specs=pl.BlockSpec((BN, OUT), lambda nb: (nb, 0)),
        out_shape=jax.ShapeDtypeStruct((N, OUT), jnp.float32),
    )


def _edge_body(p_hbm, qn_hbm, kn_hbm, xw_hbm, acc_hbm, ext_hbm,
               packed, iqb, isb, ikb, dstb, qa, ka, wv, exb, rows2,
               acc_sh, ext_sh,
               sem_q0, sem_q1, sem_k0, sem_k1, sem_r0, sem_r1,
               sem_s0, sem_s1, sem_e0, sem_e1):
    c = lax.axis_index("c")
    s = lax.axis_index("s")
    wid = s * NC + c
    brow0 = wid * NB
    row0 = s * ROWS_PER_SUB

    for sl in range(2):
        def zb(b, cc):
            for j in range(H // 16):
                rows2[sl, b, pl.ds(j * 16, 16)] = jnp.zeros((16,), jnp.float32)
            return cc
        lax.fori_loop(0, B, zb, 0)
    for sl in range(2):
        def ze(b, cc):
            exb[sl, b, pl.ds(0, 16)] = jnp.zeros((16,), jnp.float32)
            return cc
        lax.fori_loop(0, B, ze, 0)
    zcopies = []
    for k in range(ROWS_PER_SUB // B):
        zcopies.append(pltpu.async_copy(
            rows2.at[0], acc_sh.at[pl.ds(row0 + k * B, B)], sem_s0))
        zcopies.append(pltpu.async_copy(
            exb.at[0], ext_sh.at[pl.ds(row0 + k * B, B)], sem_e0))
    rem = ROWS_PER_SUB % B
    zcopies.append(pltpu.async_copy(
        rows2.at[0].at[pl.ds(0, rem)],
        acc_sh.at[pl.ds(row0 + (ROWS_PER_SUB // B) * B, rem)], sem_s0))
    zcopies.append(pltpu.async_copy(
        exb.at[0].at[pl.ds(0, rem)],
        ext_sh.at[pl.ds(row0 + (ROWS_PER_SUB // B) * B, rem)], sem_e0))
    zcopies.append(pltpu.async_copy(p_hbm.at[pl.ds(brow0, NB)], packed, sem_q0))
    @pl.when(s == 0)
    def _zero_tail():
        pltpu.sync_copy(rows2.at[0].at[pl.ds(0, TAIL_ROWS)],
                        acc_sh.at[pl.ds(TAIL_ROW0, TAIL_ROWS)])
        pltpu.sync_copy(exb.at[0].at[pl.ds(0, TAIL_ROWS)],
                        ext_sh.at[pl.ds(TAIL_ROW0, TAIL_ROWS)])
    for cp in zcopies:
        cp.wait()
    plsc.subcore_barrier()

    sem_q = (sem_q0, sem_q1)
    sem_k = (sem_k0, sem_k1)
    sem_r = (sem_r0, sem_r1)
    sem_s = (sem_s0, sem_s1)
    sem_e = (sem_e0, sem_e1)

    def fire(t, slot):
        @pl.when(t >= 2)
        def _drain():
            pltpu.make_async_copy(rows2.at[slot], acc_sh.at[dstb.at[slot]],
                                  sem_s[slot]).wait()
            pltpu.make_async_copy(exb.at[slot], ext_sh.at[dstb.at[slot]],
                                  sem_e[slot]).wait()
        for i in range(B // 16):
            sl16 = pl.ds(i * 16, 16)
            pch = packed[t, sl16]
            iqc = lax.shift_right_logical(pch, 14)
            srcv = jnp.bitwise_and(pch, 16383)
            etc = jnp.bitwise_and(iqc, R - 1)
            iqb[slot, sl16] = iqc
            dstb[slot, sl16] = lax.shift_right_logical(pch, 17)
            isb[slot, sl16] = srcv * R + etc
            ikb[slot, sl16] = etc * N + srcv
        pltpu.async_copy(qn_hbm.at[iqb.at[slot]], qa.at[slot], sem_q[slot])
        pltpu.async_copy(kn_hbm.at[isb.at[slot]], ka.at[slot], sem_k[slot])
        pltpu.async_copy(xw_hbm.at[ikb.at[slot]], rows2.at[slot], sem_r[slot])

    def process(t, slot):
        pltpu.make_async_copy(qn_hbm.at[iqb.at[slot]], qa.at[slot],
                              sem_q[slot]).wait()
        pltpu.make_async_copy(kn_hbm.at[isb.at[slot]], ka.at[slot],
                              sem_k[slot]).wait()
        rr = rows2.at[slot]
        for i in range(B // 16):
            sl16 = pl.ds(i * 16, 16)
            bidx = lax.iota(jnp.int32, 16) + i * 16
            z = qa[slot, sl16] + ka[slot, sl16]
            z = jnp.maximum(z, 0.2 * z)
            w = jnp.exp(z)
            wv[sl16] = w
            plsc.store_scatter(exb.at[slot], [bidx, jnp.zeros((16,), jnp.int32)], w)
        pltpu.make_async_copy(xw_hbm.at[ikb.at[slot]], rows2.at[slot],
                              sem_r[slot]).wait()
        @plsc.parallel_loop(0, B, 1, unroll=16)
        def mulb(b):
            wb = plsc.load_gather(wv, [jnp.full((16,), 0, jnp.int32) + b])
            for j in range(H // 16):
                rr[b, pl.ds(j * 16, 16)] = rr[b, pl.ds(j * 16, 16)] * wb
        pltpu.async_copy(rr, acc_sh.at[dstb.at[slot]], sem_s[slot], add=True)
        pltpu.async_copy(exb.at[slot], ext_sh.at[dstb.at[slot]], sem_e[slot],
                         add=True)

    fire(0, 0)
    def body(u, carry):
        t0 = 2 * u
        @pl.when(t0 + 1 < NB)
        def _f1():
            fire(t0 + 1, 1)
        process(t0, 0)
        @pl.when(t0 + 2 < NB)
        def _f0():
            fire(t0 + 2, 0)
        @pl.when(t0 + 1 < NB)
        def _p1():
            process(t0 + 1, 1)
        return carry
    lax.fori_loop(0, (NB + 1) // 2, body, 0)
    for slot in range(2):
        pltpu.make_async_copy(rows2.at[slot], acc_sh.at[dstb.at[slot]],
                              sem_s[slot]).wait()
        pltpu.make_async_copy(exb.at[slot], ext_sh.at[dstb.at[slot]],
                              sem_e[slot]).wait()
    plsc.subcore_barrier()

    pltpu.sync_copy(acc_sh.at[pl.ds(row0, ROWS_PER_SUB)],
                    acc_hbm.at[c, pl.ds(row0, ROWS_PER_SUB)])
    pltpu.sync_copy(ext_sh.at[pl.ds(row0, ROWS_PER_SUB)],
                    ext_hbm.at[c, pl.ds(row0, ROWS_PER_SUB)])
    @pl.when(s == 0)
    def _dump_tail():
        pltpu.sync_copy(acc_sh.at[pl.ds(TAIL_ROW0, TAIL_ROWS)],
                        acc_hbm.at[c, pl.ds(TAIL_ROW0, TAIL_ROWS)])
        pltpu.sync_copy(ext_sh.at[pl.ds(TAIL_ROW0, TAIL_ROWS)],
                        ext_hbm.at[c, pl.ds(TAIL_ROW0, TAIL_ROWS)])


def _make_edge():
    mesh = plsc.VectorSubcoreMesh(core_axis_name="c", subcore_axis_name="s")
    return functools.partial(
        pl.kernel,
        out_type=[
            jax.ShapeDtypeStruct((NC, N, H), jnp.float32),
            jax.ShapeDtypeStruct((NC, N, EW), jnp.float32),
        ],
        mesh=mesh,
        compiler_params=pltpu.CompilerParams(use_tc_tiling_on_sc=False,
                                             needs_layout_passes=False),
        scratch_types=[
            pltpu.VMEM((NB, B), jnp.int32),
            pltpu.VMEM((2, B), jnp.int32),
            pltpu.VMEM((2, B), jnp.int32),
            pltpu.VMEM((2, B), jnp.int32),
            pltpu.VMEM((2, B), jnp.int32),
            pltpu.VMEM((2, B), jnp.float32),
            pltpu.VMEM((2, B), jnp.float32),
            pltpu.VMEM((B,), jnp.float32),
            pltpu.VMEM((2, B, EW), jnp.float32),
            pltpu.VMEM((2, B, H), jnp.float32),
            pltpu.VMEM_SHARED((N, H), jnp.float32),
            pltpu.VMEM_SHARED((N, EW), jnp.float32),
        ] + [pltpu.SemaphoreType.DMA] * 10,
    )(_edge_body)


def kernel(x, edge_index, edge_type, W1, Q1, K1, W2, Q2, K2, Wl, bl):
    src = edge_index[0].reshape(625, 512)
    dst = edge_index[1].reshape(625, 512)
    et = edge_type.reshape(625, 512)
    packed = _make_idx()(src, dst, et).reshape(E // B, B)

    dense = _make_dense()
    edge = _make_edge()
    combine = _make_combine()

    xw1, qn1, kn1 = dense(x, W1, Q1.reshape(R, 1, H), K1.reshape(R, 1, H))
    acc1, ext1 = edge(packed, qn1.reshape(N * R), kn1.reshape(N * R), xw1)
    h1 = combine(acc1, ext1)

    xw2, qn2, kn2 = dense(h1, W2, Q2.reshape(R, 1, H), K2.reshape(R, 1, H))
    acc2, ext2 = edge(packed, qn2.reshape(N * R), kn2.reshape(N * R), xw2)
    return _make_final()(acc2, ext2, Wl, bl.reshape(1, OUT))
